# baseline (speedup 1.0000x reference)
import functools

import jax
import jax.numpy as jnp
from jax import lax
from jax.experimental import pallas as pl
from jax.experimental.pallas import tpu as pltpu
from jax.experimental.pallas import tpu_sc as plsc

NU = 25000
NI = 25000
NN = NU + NI
D = 64
E = 800000

NC = 2
NS = 16
CH = 128
KS = 1
NSUPER = 400
NJ = NSUPER // 2
NCHUNK = NSUPER * KS
EPT = NCHUNK * CH
EPAD = EPT * NS
NCROW = EPAD // CH
DH = 64
ROWS_PAD = 25088
STRIPE = ROWS_PAD // NS
ABSORB = 25080

DEG_PER_W = (2 * E) // (NC * NS)
CNT_WORDS = 51200

_MESH = plsc.VectorSubcoreMesh(
    core_axis_name="c", subcore_axis_name="s", num_cores=NC, num_subcores=NS)


def _wid():
    return lax.axis_index("s") * NC + lax.axis_index("c")


def _deg_body(allidx_hbm, out_hbm, cnt, idxbuf):
    wid = _wid()
    zeros16 = jnp.zeros((16,), jnp.float32)
    ones16 = jnp.ones((16,), jnp.float32)

    def zero_body(i, c):
        cnt[pl.ds(i * 16, 16)] = zeros16
        return c
    lax.fori_loop(0, CNT_WORDS // 16, zero_body, 0)

    pltpu.sync_copy(allidx_hbm.at[pl.ds(wid * DEG_PER_W, DEG_PER_W)], idxbuf)

    def body(i, c):
        iv = idxbuf[pl.ds(i * 16, 16)]
        plsc.addupdate_scatter(cnt, [iv], ones16)
        return c
    lax.fori_loop(0, DEG_PER_W // 16, body, 0)

    pltpu.sync_copy(cnt, out_hbm.at[wid])


_deg_kernel = functools.partial(
    pl.kernel,
    out_type=jax.ShapeDtypeStruct((NC * NS, CNT_WORDS), jnp.float32),
    mesh=_MESH,
    compiler_params=pltpu.CompilerParams(needs_layout_passes=False),
    scratch_types=[
        pltpu.VMEM((CNT_WORDS,), jnp.float32),
        pltpu.VMEM((DEG_PER_W,), jnp.int32),
    ],
)(_deg_body)


def _seg_body(zeros_hbm,
              s0, d0, t0, s1, d1, t1, s2, d2, t2, s3, d3, t3,
              o0, o1, o2, o3,
              acc, sidx, didx, rows, gsem, ssem):
    c = lax.axis_index("c")
    sid = lax.axis_index("s")
    r0 = sid * STRIPE

    def run(src, dst, tab, out):
        crow = sid * NCHUNK

        def load_idx(sup, p):
            pltpu.sync_copy(src.at[pl.ds(crow + sup * KS, KS)], sidx.at[p])
            pltpu.sync_copy(dst.at[pl.ds(crow + sup * KS, KS)], didx.at[p])

        def fire_gathers(p):
            for k in range(KS):
                pltpu.async_copy(tab.at[sidx.at[p, k]], rows.at[p, k], gsem)

        def drain_gathers(p):
            for k in range(KS):
                pltpu.make_async_copy(tab.at[sidx.at[p, k]],
                                      rows.at[p, k], gsem).wait()

        def fire_scatters(p):
            for k in range(KS):
                pltpu.async_copy(rows.at[p, k], acc.at[didx.at[p, k]],
                                 ssem, add=True)

        def drain_scatters(p):
            for k in range(KS):
                pltpu.make_async_copy(rows.at[p, k],
                                      acc.at[didx.at[p, k]], ssem).wait()

        load_idx(0, 0)
        fire_gathers(0)
        pltpu.sync_copy(zeros_hbm.at[pl.ds(r0, STRIPE)],
                        acc.at[pl.ds(r0, STRIPE)])
        plsc.subcore_barrier()

        def body(j, carry):
            @pl.when(j > 0)
            def _():
                drain_scatters(1)
            load_idx(2 * j + 1, 1)
            fire_gathers(1)
            drain_gathers(0)
            fire_scatters(0)
            drain_scatters(0)

            @pl.when(j < NJ - 1)
            def _():
                load_idx(2 * j + 2, 0)
                fire_gathers(0)
            drain_gathers(1)
            fire_scatters(1)
            return carry
        lax.fori_loop(0, NJ, body, 0)
        drain_scatters(1)
        plsc.subcore_barrier()
        pltpu.sync_copy(acc.at[pl.ds(r0, STRIPE)], out.at[pl.ds(r0, STRIPE)])

    @pl.when(c == 0)
    def _():
        run(s0, d0, t0, o0)
        run(s1, d1, t1, o1)

    @pl.when(c == 1)
    def _():
        run(s2, d2, t2, o2)
        run(s3, d3, t3, o3)


_OUT4 = tuple(jax.ShapeDtypeStruct((ROWS_PAD, DH), jnp.float32)
              for _ in range(4))

_seg_kernel = functools.partial(
    pl.kernel,
    out_type=_OUT4,
    mesh=_MESH,
    compiler_params=pltpu.CompilerParams(use_tc_tiling_on_sc=False),
    scratch_types=[
        pltpu.MemorySpace.VMEM_SHARED((ROWS_PAD, DH), jnp.float32),
        pltpu.VMEM((2, KS, CH), jnp.int32),
        pltpu.VMEM((2, KS, CH), jnp.int32),
        pltpu.VMEM((2, KS, CH, DH), jnp.float32),
        pltpu.SemaphoreType.DMA,
        pltpu.SemaphoreType.DMA,
    ],
)(_seg_body)


def _deg_finish_body(cnt_ref, dinv_ref):
    c = jnp.sum(cnt_ref[...], axis=0)
    dinv_ref[...] = lax.rsqrt(2.0 * c)


def _deg_finish(cnt):
    return pl.pallas_call(
        _deg_finish_body,
        out_shape=jax.ShapeDtypeStruct((CNT_WORDS // 128, 128), jnp.float32),
    )(cnt)


_RB = 1000


def _pre_body(ue_ref, ie_ref, vf_ref, tf_ref, su_ref, si_ref,
              wv_ref, bv_ref, wt_ref, bt_ref,
              vd_ref, td_ref, zu_ref, z0_ref, z1_ref, z2_ref):
    vf = vf_ref[...]
    tf = tf_ref[...]
    vn = vf * lax.rsqrt(jnp.maximum(jnp.sum(vf * vf, axis=1, keepdims=True),
                                    1e-24))
    tn = tf * lax.rsqrt(jnp.maximum(jnp.sum(tf * tf, axis=1, keepdims=True),
                                    1e-24))
    vd = lax.dot_general(vn, wv_ref[...], (((1,), (1,)), ((), ())),
                         preferred_element_type=jnp.float32) + bv_ref[...]
    td = lax.dot_general(tn, wt_ref[...], (((1,), (1,)), ((), ())),
                         preferred_element_type=jnp.float32) + bt_ref[...]
    su = su_ref[...]
    si = si_ref[...]
    vd_ref[...] = vd
    td_ref[...] = td
    zu_ref[...] = su * ue_ref[...]
    z0_ref[...] = si * ie_ref[...]
    z1_ref[...] = si * vd
    z2_ref[...] = si * td


def _pre(user_emb, item_emb, v_feat, t_feat, s_u, s_i, Wv, bv, Wt, bt):
    grid = (NU // _RB,)
    rb = lambda i: (i, 0)
    full = lambda i: (0, 0)
    out_shapes = tuple(jax.ShapeDtypeStruct((NU, D), jnp.float32)
                       for _ in range(6))
    return pl.pallas_call(
        _pre_body,
        grid=grid,
        in_specs=[
            pl.BlockSpec((_RB, D), rb), pl.BlockSpec((_RB, D), rb),
            pl.BlockSpec((_RB, 128), rb), pl.BlockSpec((_RB, 128), rb),
            pl.BlockSpec((_RB, 1), rb), pl.BlockSpec((_RB, 1), rb),
            pl.BlockSpec((D, 128), full), pl.BlockSpec((1, D), full),
            pl.BlockSpec((D, 128), full), pl.BlockSpec((1, D), full),
        ],
        out_specs=tuple(pl.BlockSpec((_RB, D), rb) for _ in range(6)),
        out_shape=out_shapes,
    )(user_emb, item_emb, v_feat, t_feat, s_u, s_i, Wv, bv, Wt, bt)


def _mid_body(u0_ref, u1_ref, u2_ref, ui_ref, su_ref, si_ref,
              z0_ref, z1_ref, z2_ref, zi_ref):
    su2 = jnp.square(su_ref[...])
    si2 = jnp.square(si_ref[...])
    z0_ref[...] = su2 * u0_ref[...]
    z1_ref[...] = su2 * u1_ref[...]
    z2_ref[...] = su2 * u2_ref[...]
    zi_ref[...] = si2 * ui_ref[...]


def _mid(u0, u1, u2, ui, s_u, s_i):
    grid = (NU // _RB,)
    rb = lambda i: (i, 0)
    return pl.pallas_call(
        _mid_body,
        grid=grid,
        in_specs=[pl.BlockSpec((_RB, D), rb)] * 4
        + [pl.BlockSpec((_RB, 1), rb)] * 2,
        out_specs=tuple(pl.BlockSpec((_RB, D), rb) for _ in range(4)),
        out_shape=tuple(jax.ShapeDtypeStruct((NU, D), jnp.float32)
                        for _ in range(4)),
    )(u0, u1, u2, ui, s_u, s_i)


def _final_body(x0_ref, x1_ref, x2_ref, a0_ref, a1_ref, a2_ref,
                b0_ref, b1_ref, b2_ref, s_ref, w_ref, bias_ref, out_ref):
    s = s_ref[...]
    m0 = (x0_ref[...] + s * (a0_ref[...] + b0_ref[...])) * (1.0 / 3.0)
    m1 = (x1_ref[...] + s * (a1_ref[...] + b1_ref[...])) * (1.0 / 3.0)
    m2 = (x2_ref[...] + s * (a2_ref[...] + b2_ref[...])) * (1.0 / 3.0)
    m = jnp.concatenate([m0, m1, m2], axis=1)
    out_ref[...] = lax.dot_general(
        m, w_ref[...], (((1,), (1,)), ((), ())),
        preferred_element_type=jnp.float32) + bias_ref[...]


def _final(xs, u1s, u2s, s, W, b):
    grid = (NU // _RB,)
    rb = lambda i: (i, 0)
    full = lambda i: (0, 0)
    return pl.pallas_call(
        _final_body,
        grid=grid,
        in_specs=[pl.BlockSpec((_RB, D), rb)] * 9
        + [pl.BlockSpec((_RB, 1), rb),
           pl.BlockSpec((D, 3 * D), full), pl.BlockSpec((1, D), full)],
        out_specs=pl.BlockSpec((_RB, D), rb),
        out_shape=jax.ShapeDtypeStruct((NU, D), jnp.float32),
    )(*xs, *u1s, *u2s, s, W, b)


def _pipeline(user_emb, item_emb, v_feat, t_feat, Wv, bv, Wt, bt,
              Wu, bu, Wi, bi, edge_index):
    row = edge_index[0]
    colL = edge_index[1] - NU
    bv = bv.reshape(1, D)
    bt = bt.reshape(1, D)
    bu = bu.reshape(1, D)
    bi = bi.reshape(1, D)

    pad_src = jnp.zeros((EPAD - E,), jnp.int32)
    pad_dst = jnp.full((EPAD - E,), ABSORB, jnp.int32)
    row_src = jnp.concatenate([row, pad_src]).reshape(NCROW, CH)
    row_dst = jnp.concatenate([row, pad_dst]).reshape(NCROW, CH)
    colL_src = jnp.concatenate([colL, pad_src]).reshape(NCROW, CH)
    colL_dst = jnp.concatenate([colL, pad_dst]).reshape(NCROW, CH)

    allidx = jnp.concatenate([row, colL + NU])
    cnt = _deg_kernel(allidx)
    dinv = _deg_finish(cnt.reshape(NC * NS, CNT_WORDS // 128, 128))
    s = dinv.reshape(-1)[:NN]
    s_u = s[:NU].reshape(NU, 1)
    s_i = s[NU:].reshape(NI, 1)

    v_dense, t_dense, Zu0, Zi0_0, Zi0_1, Zi0_2 = _pre(
        user_emb, item_emb, v_feat, t_feat, s_u, s_i, Wv, bv, Wt, bt)

    zeros = jnp.zeros((ROWS_PAD, DH), jnp.float32)

    def seg4(sd0, t0, sd1, t1, sd2, t2, sd3, t3):
        outs = _seg_kernel(
            zeros,
            sd0[0], sd0[1], t0, sd1[0], sd1[1], t1,
            sd2[0], sd2[1], t2, sd3[0], sd3[1], t3)
        return tuple(o[:NU] for o in outs)

    iu = (colL_src, row_dst)
    ui = (row_src, colL_dst)

    U1u0, U1u1, U1u2, U1i = seg4(iu, Zi0_0, iu, Zi0_1, iu, Zi0_2, ui, Zu0)

    Z1u0, Z1u1, Z1u2, Z1i = _mid(U1u0, U1u1, U1u2, U1i, s_u, s_i)

    U2u, U2i0, U2i1, U2i2 = seg4(iu, Z1i, ui, Z1u0, ui, Z1u1, ui, Z1u2)

    user = _final((user_emb, user_emb, user_emb),
                  (U1u0, U1u1, U1u2),
                  (U2u, U2u, U2u), s_u, Wu, bu)
    item = _final((item_emb, v_dense, t_dense),
                  (U1i, U1i, U1i),
                  (U2i0, U2i1, U2i2), s_i, Wi, bi)
    return (user, item)


def kernel(user_emb, item_emb, v_feat, t_feat, Wv, bv, Wt, bt,
           Wu, bu, Wi, bi, edge_index):
    return _pipeline(user_emb, item_emb, v_feat, t_feat, Wv, bv, Wt, bt,
                     Wu, bu, Wi, bi, edge_index)


# speedup vs baseline: 1.1200x; 1.1200x over previous
import functools

import jax
import jax.numpy as jnp
from jax import lax
from jax.experimental import pallas as pl
from jax.experimental.pallas import tpu as pltpu
from jax.experimental.pallas import tpu_sc as plsc

NU = 25000
NI = 25000
NN = NU + NI
D = 64
E = 800000

NC = 2
NS = 16
CH = 128
KS = 8
NSUPER = 50
NJ = NSUPER // 2
NCHUNK = NSUPER * KS
EPT = NCHUNK * CH
EPAD = EPT * NS
NCROW = EPAD // CH
DH = 32
ROWS_PAD = 25088
STRIPE = ROWS_PAD // NS
ABSORB = 25080

EPW = E // (NC * NS)
CNT_WORDS = 51200

_MESH = plsc.VectorSubcoreMesh(
    core_axis_name="c", subcore_axis_name="s", num_cores=NC, num_subcores=NS)


def _wid():
    return lax.axis_index("s") * NC + lax.axis_index("c")


def _deg_body(row_hbm, col_hbm, out_hbm, cnt, idxbuf):
    wid = _wid()
    zeros16 = jnp.zeros((16,), jnp.float32)
    ones16 = jnp.ones((16,), jnp.float32)

    def zero_body(i, c):
        cnt[pl.ds(i * 16, 16)] = zeros16
        return c
    lax.fori_loop(0, CNT_WORDS // 16, zero_body, 0)

    pltpu.sync_copy(row_hbm.at[pl.ds(wid * EPW, EPW)], idxbuf.at[pl.ds(0, EPW)])
    pltpu.sync_copy(col_hbm.at[pl.ds(wid * EPW, EPW)],
                    idxbuf.at[pl.ds(EPW, EPW)])

    def body(i, c):
        iv = idxbuf[pl.ds(i * 16, 16)]
        plsc.addupdate_scatter(cnt, [iv], ones16)
        return c
    lax.fori_loop(0, 2 * EPW // 16, body, 0)

    pltpu.sync_copy(cnt, out_hbm.at[wid])


_deg_kernel = functools.partial(
    pl.kernel,
    out_type=jax.ShapeDtypeStruct((NC * NS, CNT_WORDS), jnp.float32),
    mesh=_MESH,
    compiler_params=pltpu.CompilerParams(needs_layout_passes=False),
    scratch_types=[
        pltpu.VMEM((CNT_WORDS,), jnp.float32),
        pltpu.VMEM((2 * EPW,), jnp.int32),
    ],
)(_deg_body)


def _seg_body(zeros_hbm,
              s0, d0, t0a, t0b, s1, d1, t1a, t1b,
              s2, d2, t2a, t2b, s3, d3, t3a, t3b,
              o0a, o0b, o1a, o1b, o2a, o2b, o3a, o3b,
              acc, sidx, didx, rows, gsem, ssem):
    c = lax.axis_index("c")
    sid = lax.axis_index("s")
    r0 = sid * STRIPE

    def run(src, dst, tab, out):
        crow = sid * NCHUNK

        def load_idx(sup, p):
            pltpu.sync_copy(src.at[pl.ds(crow + sup * KS, KS)], sidx.at[p])
            pltpu.sync_copy(dst.at[pl.ds(crow + sup * KS, KS)], didx.at[p])

        def fire_gathers(p):
            for k in range(KS):
                pltpu.async_copy(tab.at[sidx.at[p, k]], rows.at[p, k], gsem)

        def drain_gathers(p):
            for k in range(KS):
                pltpu.make_async_copy(tab.at[sidx.at[p, k]],
                                      rows.at[p, k], gsem).wait()

        def fire_scatters(p):
            for k in range(KS):
                pltpu.async_copy(rows.at[p, k], acc.at[didx.at[p, k]],
                                 ssem, add=True)

        def drain_scatters(p):
            for k in range(KS):
                pltpu.make_async_copy(rows.at[p, k],
                                      acc.at[didx.at[p, k]], ssem).wait()

        load_idx(0, 0)
        fire_gathers(0)
        pltpu.sync_copy(zeros_hbm.at[pl.ds(r0, STRIPE)],
                        acc.at[pl.ds(r0, STRIPE)])
        plsc.subcore_barrier()

        def body(j, carry):
            @pl.when(j > 0)
            def _():
                drain_scatters(1)
            load_idx(2 * j + 1, 1)
            fire_gathers(1)
            drain_gathers(0)
            fire_scatters(0)
            drain_scatters(0)

            @pl.when(j < NJ - 1)
            def _():
                load_idx(2 * j + 2, 0)
                fire_gathers(0)
            drain_gathers(1)
            fire_scatters(1)
            return carry
        lax.fori_loop(0, NJ, body, 0)
        drain_scatters(1)
        plsc.subcore_barrier()
        pltpu.sync_copy(acc.at[pl.ds(r0, STRIPE)], out.at[pl.ds(r0, STRIPE)])

    @pl.when(c == 0)
    def _():
        run(s0, d0, t0a, o0a)
        run(s0, d0, t0b, o0b)
        run(s1, d1, t1a, o1a)
        run(s1, d1, t1b, o1b)

    @pl.when(c == 1)
    def _():
        run(s2, d2, t2a, o2a)
        run(s2, d2, t2b, o2b)
        run(s3, d3, t3a, o3a)
        run(s3, d3, t3b, o3b)


_OUT8 = tuple(jax.ShapeDtypeStruct((ROWS_PAD, DH), jnp.float32)
              for _ in range(8))

_seg_kernel = functools.partial(
    pl.kernel,
    out_type=_OUT8,
    mesh=_MESH,
    compiler_params=pltpu.CompilerParams(use_tc_tiling_on_sc=False),
    scratch_types=[
        pltpu.MemorySpace.VMEM_SHARED((ROWS_PAD, DH), jnp.float32),
        pltpu.VMEM((2, KS, CH), jnp.int32),
        pltpu.VMEM((2, KS, CH), jnp.int32),
        pltpu.VMEM((2, KS, CH, DH), jnp.float32),
        pltpu.SemaphoreType.DMA,
        pltpu.SemaphoreType.DMA,
    ],
)(_seg_body)


def _deg_finish_body(cnt_ref, dinv_ref):
    c = jnp.sum(cnt_ref[...], axis=0)
    dinv_ref[...] = lax.rsqrt(2.0 * c)


def _deg_finish(cnt):
    return pl.pallas_call(
        _deg_finish_body,
        out_shape=jax.ShapeDtypeStruct((CNT_WORDS // 128, 128), jnp.float32),
    )(cnt)


_RB = 1000


def _pre_body(ue_ref, ie_ref, vf_ref, tf_ref, su_ref, si_ref,
              wv_ref, bv_ref, wt_ref, bt_ref,
              vd_ref, td_ref,
              zua_ref, zub_ref, z0a_ref, z0b_ref,
              z1a_ref, z1b_ref, z2a_ref, z2b_ref):
    vf = vf_ref[...]
    tf = tf_ref[...]
    vn = vf * lax.rsqrt(jnp.maximum(jnp.sum(vf * vf, axis=1, keepdims=True),
                                    1e-24))
    tn = tf * lax.rsqrt(jnp.maximum(jnp.sum(tf * tf, axis=1, keepdims=True),
                                    1e-24))
    vd = lax.dot_general(vn, wv_ref[...], (((1,), (1,)), ((), ())),
                         preferred_element_type=jnp.float32) + bv_ref[...]
    td = lax.dot_general(tn, wt_ref[...], (((1,), (1,)), ((), ())),
                         preferred_element_type=jnp.float32) + bt_ref[...]
    su = su_ref[...]
    si = si_ref[...]
    zu = su * ue_ref[...]
    z0 = si * ie_ref[...]
    z1 = si * vd
    z2 = si * td
    vd_ref[...] = vd
    td_ref[...] = td
    zua_ref[...] = zu[:, :DH]
    zub_ref[...] = zu[:, DH:]
    z0a_ref[...] = z0[:, :DH]
    z0b_ref[...] = z0[:, DH:]
    z1a_ref[...] = z1[:, :DH]
    z1b_ref[...] = z1[:, DH:]
    z2a_ref[...] = z2[:, :DH]
    z2b_ref[...] = z2[:, DH:]


def _pre(user_emb, item_emb, v_feat, t_feat, s_u, s_i, Wv, bv, Wt, bt):
    grid = (NU // _RB,)
    rb = lambda i: (i, 0)
    full = lambda i: (0, 0)
    out_shapes = (jax.ShapeDtypeStruct((NU, D), jnp.float32),
                  jax.ShapeDtypeStruct((NU, D), jnp.float32)) + tuple(
        jax.ShapeDtypeStruct((NU, DH), jnp.float32) for _ in range(8))
    return pl.pallas_call(
        _pre_body,
        grid=grid,
        in_specs=[
            pl.BlockSpec((_RB, D), rb), pl.BlockSpec((_RB, D), rb),
            pl.BlockSpec((_RB, 128), rb), pl.BlockSpec((_RB, 128), rb),
            pl.BlockSpec((_RB, 1), rb), pl.BlockSpec((_RB, 1), rb),
            pl.BlockSpec((D, 128), full), pl.BlockSpec((1, D), full),
            pl.BlockSpec((D, 128), full), pl.BlockSpec((1, D), full),
        ],
        out_specs=(pl.BlockSpec((_RB, D), rb), pl.BlockSpec((_RB, D), rb))
        + tuple(pl.BlockSpec((_RB, DH), rb) for _ in range(8)),
        out_shape=out_shapes,
    )(user_emb, item_emb, v_feat, t_feat, s_u, s_i, Wv, bv, Wt, bt)


_MB = 1568


def _mid_body(u0a_ref, u0b_ref, u1a_ref, u1b_ref, u2a_ref, u2b_ref,
              uia_ref, uib_ref, su_ref, si_ref,
              z0a_ref, z0b_ref, z1a_ref, z1b_ref, z2a_ref, z2b_ref,
              zia_ref, zib_ref):
    su2 = jnp.square(su_ref[...])
    si2 = jnp.square(si_ref[...])
    z0a_ref[...] = su2 * u0a_ref[...]
    z0b_ref[...] = su2 * u0b_ref[...]
    z1a_ref[...] = su2 * u1a_ref[...]
    z1b_ref[...] = su2 * u1b_ref[...]
    z2a_ref[...] = su2 * u2a_ref[...]
    z2b_ref[...] = su2 * u2b_ref[...]
    zia_ref[...] = si2 * uia_ref[...]
    zib_ref[...] = si2 * uib_ref[...]


def _mid(u0, u1, u2, ui, su_pad, si_pad):
    grid = (ROWS_PAD // _MB,)
    rb = lambda i: (i, 0)
    return pl.pallas_call(
        _mid_body,
        grid=grid,
        in_specs=[pl.BlockSpec((_MB, DH), rb)] * 8
        + [pl.BlockSpec((_MB, 1), rb)] * 2,
        out_specs=tuple(pl.BlockSpec((_MB, DH), rb) for _ in range(8)),
        out_shape=tuple(jax.ShapeDtypeStruct((ROWS_PAD, DH), jnp.float32)
                        for _ in range(8)),
    )(*u0, *u1, *u2, *ui, su_pad, si_pad)


def _final_body(x0_ref, x1_ref, x2_ref,
                a0a_ref, a0b_ref, a1a_ref, a1b_ref, a2a_ref, a2b_ref,
                b0a_ref, b0b_ref, b1a_ref, b1b_ref, b2a_ref, b2b_ref,
                s_ref, w_ref, bias_ref, out_ref):
    s = s_ref[...]
    third = 1.0 / 3.0
    m0a = (x0_ref[...][:, :DH] + s * (a0a_ref[...] + b0a_ref[...])) * third
    m0b = (x0_ref[...][:, DH:] + s * (a0b_ref[...] + b0b_ref[...])) * third
    m1a = (x1_ref[...][:, :DH] + s * (a1a_ref[...] + b1a_ref[...])) * third
    m1b = (x1_ref[...][:, DH:] + s * (a1b_ref[...] + b1b_ref[...])) * third
    m2a = (x2_ref[...][:, :DH] + s * (a2a_ref[...] + b2a_ref[...])) * third
    m2b = (x2_ref[...][:, DH:] + s * (a2b_ref[...] + b2b_ref[...])) * third
    m = jnp.concatenate([m0a, m0b, m1a, m1b, m2a, m2b], axis=1)
    out_ref[...] = lax.dot_general(
        m, w_ref[...], (((1,), (1,)), ((), ())),
        preferred_element_type=jnp.float32) + bias_ref[...]


def _final(xs, u1s, u2s, s, W, b):
    grid = (NU // _RB,)
    rb = lambda i: (i, 0)
    full = lambda i: (0, 0)
    halves = u1s + u2s
    return pl.pallas_call(
        _final_body,
        grid=grid,
        in_specs=[pl.BlockSpec((_RB, D), rb)] * 3
        + [pl.BlockSpec((_RB, DH), rb)] * 12
        + [pl.BlockSpec((_RB, 1), rb),
           pl.BlockSpec((D, 3 * D), full), pl.BlockSpec((1, D), full)],
        out_specs=pl.BlockSpec((_RB, D), rb),
        out_shape=jax.ShapeDtypeStruct((NU, D), jnp.float32),
    )(*xs, *halves, s, W, b)


def _pipeline(user_emb, item_emb, v_feat, t_feat, Wv, bv, Wt, bt,
              Wu, bu, Wi, bi, edge_index):
    row = edge_index[0]
    col = edge_index[1]
    colL = col - NU
    bv = bv.reshape(1, D)
    bt = bt.reshape(1, D)
    bu = bu.reshape(1, D)
    bi = bi.reshape(1, D)

    pad_src = jnp.zeros((EPAD - E,), jnp.int32)
    pad_dst = jnp.full((EPAD - E,), ABSORB, jnp.int32)
    row_src = jnp.concatenate([row, pad_src]).reshape(NCROW, CH)
    row_dst = jnp.concatenate([row, pad_dst]).reshape(NCROW, CH)
    colL_src = jnp.concatenate([colL, pad_src]).reshape(NCROW, CH)
    colL_dst = jnp.concatenate([colL, pad_dst]).reshape(NCROW, CH)

    cnt = _deg_kernel(row, col)
    dinv = _deg_finish(cnt.reshape(NC * NS, CNT_WORDS // 128, 128))
    dflat = dinv.reshape(-1)
    s_u = dflat[:NU].reshape(NU, 1)
    s_i = dflat[NU:NN].reshape(NI, 1)
    su_pad = dflat[:ROWS_PAD].reshape(ROWS_PAD, 1)
    si_pad = lax.dynamic_slice(dflat, (NU,), (ROWS_PAD,)).reshape(ROWS_PAD, 1)

    (v_dense, t_dense, zua, zub,
     z0a, z0b, z1a, z1b, z2a, z2b) = _pre(
        user_emb, item_emb, v_feat, t_feat, s_u, s_i, Wv, bv, Wt, bt)

    zeros = jnp.zeros((ROWS_PAD, DH), jnp.float32)

    def seg4(sd0, t0, sd1, t1, sd2, t2, sd3, t3):
        return _seg_kernel(
            zeros,
            sd0[0], sd0[1], t0[0], t0[1],
            sd1[0], sd1[1], t1[0], t1[1],
            sd2[0], sd2[1], t2[0], t2[1],
            sd3[0], sd3[1], t3[0], t3[1])

    iu = (colL_src, row_dst)
    ui = (row_src, colL_dst)

    (U1u0a, U1u0b, U1u1a, U1u1b,
     U1u2a, U1u2b, U1ia, U1ib) = seg4(
        iu, (z0a, z0b), iu, (z1a, z1b), iu, (z2a, z2b), ui, (zua, zub))

    (Z1u0a, Z1u0b, Z1u1a, Z1u1b,
     Z1u2a, Z1u2b, Z1ia, Z1ib) = _mid(
        (U1u0a, U1u0b), (U1u1a, U1u1b), (U1u2a, U1u2b), (U1ia, U1ib),
        su_pad, si_pad)

    (U2ua, U2ub, U2i0a, U2i0b,
     U2i1a, U2i1b, U2i2a, U2i2b) = seg4(
        iu, (Z1ia, Z1ib), ui, (Z1u0a, Z1u0b),
        ui, (Z1u1a, Z1u1b), ui, (Z1u2a, Z1u2b))

    user = _final((user_emb, user_emb, user_emb),
                  (U1u0a, U1u0b, U1u1a, U1u1b, U1u2a, U1u2b),
                  (U2ua, U2ub, U2ua, U2ub, U2ua, U2ub), s_u, Wu, bu)
    item = _final((item_emb, v_dense, t_dense),
                  (U1ia, U1ib, U1ia, U1ib, U1ia, U1ib),
                  (U2i0a, U2i0b, U2i1a, U2i1b, U2i2a, U2i2b), s_i, Wi, bi)
    return (user, item)


def kernel(user_emb, item_emb, v_feat, t_feat, Wv, bv, Wt, bt,
           Wu, bu, Wi, bi, edge_index):
    return _pipeline(user_emb, item_emb, v_feat, t_feat, Wv, bv, Wt, bt,
                     Wu, bu, Wi, bi, edge_index)


# speedup vs baseline: 1.1805x; 1.0540x over previous
import functools

import jax
import jax.numpy as jnp
from jax import lax
from jax.experimental import pallas as pl
from jax.experimental.pallas import tpu as pltpu
from jax.experimental.pallas import tpu_sc as plsc

NU = 25000
NI = 25000
NN = NU + NI
D = 64
E = 800000

NC = 2
NS = 16
CH = 128
KS = 8
NSUPER = 50
NJ = NSUPER // 2
NCHUNK = NSUPER * KS
EPT = NCHUNK * CH
EPAD = EPT * NS
NCROW = EPAD // CH
DH = 32
ROWS_PAD = 25088
STRIPE = ROWS_PAD // NS
ABSORB = 25080

EPW = E // (NC * NS)
CNT_WORDS = 51200

_MESH = plsc.VectorSubcoreMesh(
    core_axis_name="c", subcore_axis_name="s", num_cores=NC, num_subcores=NS)


def _wid():
    return lax.axis_index("s") * NC + lax.axis_index("c")


def _deg_body(row_hbm, col_hbm, out_hbm, cnt, idxbuf):
    wid = _wid()
    zeros16 = jnp.zeros((16,), jnp.float32)
    ones16 = jnp.ones((16,), jnp.float32)

    def zero_body(i, c):
        cnt[pl.ds(i * 16, 16)] = zeros16
        return c
    lax.fori_loop(0, CNT_WORDS // 16, zero_body, 0)

    pltpu.sync_copy(row_hbm.at[pl.ds(wid * EPW, EPW)], idxbuf.at[pl.ds(0, EPW)])
    pltpu.sync_copy(col_hbm.at[pl.ds(wid * EPW, EPW)],
                    idxbuf.at[pl.ds(EPW, EPW)])

    def body(i, c):
        iv = idxbuf[pl.ds(i * 16, 16)]
        plsc.addupdate_scatter(cnt, [iv], ones16)
        return c
    lax.fori_loop(0, 2 * EPW // 16, body, 0)

    pltpu.sync_copy(cnt, out_hbm.at[wid])


_deg_kernel = functools.partial(
    pl.kernel,
    out_type=jax.ShapeDtypeStruct((NC * NS, CNT_WORDS), jnp.float32),
    mesh=_MESH,
    compiler_params=pltpu.CompilerParams(needs_layout_passes=False),
    scratch_types=[
        pltpu.VMEM((CNT_WORDS,), jnp.float32),
        pltpu.VMEM((2 * EPW,), jnp.int32),
    ],
)(_deg_body)


def _seg_body(zeros_hbm,
              s0, d0, t0a, t0b, s1, d1, t1a, t1b,
              s2, d2, t2a, t2b, s3, d3, t3a, t3b,
              o0a, o0b, o1a, o1b, o2a, o2b, o3a, o3b,
              acc, sidx, didx, rows, gsem, ssem):
    c = lax.axis_index("c")
    sid = lax.axis_index("s")
    r0 = sid * STRIPE

    def run(src, dst, tab, out):
        crow = sid * NCHUNK

        def load_idx(sup, p):
            pltpu.sync_copy(src.at[pl.ds(crow + sup * KS, KS)], sidx.at[p])
            pltpu.sync_copy(dst.at[pl.ds(crow + sup * KS, KS)], didx.at[p])

        def fire_gathers(p):
            for k in range(KS):
                pltpu.async_copy(tab.at[sidx.at[p, k]], rows.at[p, k], gsem)

        def drain_gathers(p):
            for k in range(KS):
                pltpu.make_async_copy(tab.at[sidx.at[p, k]],
                                      rows.at[p, k], gsem).wait()

        def fire_scatters(p):
            for k in range(KS):
                pltpu.async_copy(rows.at[p, k], acc.at[didx.at[p, k]],
                                 ssem, add=True)

        def drain_scatters(p):
            for k in range(KS):
                pltpu.make_async_copy(rows.at[p, k],
                                      acc.at[didx.at[p, k]], ssem).wait()

        load_idx(0, 0)
        fire_gathers(0)
        pltpu.sync_copy(zeros_hbm.at[pl.ds(r0, STRIPE)],
                        acc.at[pl.ds(r0, STRIPE)])
        plsc.subcore_barrier()

        def body(j, carry):
            @pl.when(j > 0)
            def _():
                drain_scatters(1)
            load_idx(2 * j + 1, 1)
            fire_gathers(1)
            drain_gathers(0)
            fire_scatters(0)
            drain_scatters(0)

            @pl.when(j < NJ - 1)
            def _():
                load_idx(2 * j + 2, 0)
                fire_gathers(0)
            drain_gathers(1)
            fire_scatters(1)
            return carry
        lax.fori_loop(0, NJ, body, 0)
        drain_scatters(1)
        plsc.subcore_barrier()
        pltpu.sync_copy(acc.at[pl.ds(r0, STRIPE)], out.at[pl.ds(r0, STRIPE)])

    @pl.when(c == 0)
    def _():
        run(s0, d0, t0a, o0a)
        run(s0, d0, t0b, o0b)
        run(s1, d1, t1a, o1a)
        run(s1, d1, t1b, o1b)

    @pl.when(c == 1)
    def _():
        run(s2, d2, t2a, o2a)
        run(s2, d2, t2b, o2b)
        run(s3, d3, t3a, o3a)
        run(s3, d3, t3b, o3b)


_OUT8 = tuple(jax.ShapeDtypeStruct((ROWS_PAD, DH), jnp.float32)
              for _ in range(8))

_seg_kernel = functools.partial(
    pl.kernel,
    out_type=_OUT8,
    mesh=_MESH,
    compiler_params=pltpu.CompilerParams(use_tc_tiling_on_sc=False),
    scratch_types=[
        pltpu.MemorySpace.VMEM_SHARED((ROWS_PAD, DH), jnp.float32),
        pltpu.VMEM((2, KS, CH), jnp.int32),
        pltpu.VMEM((2, KS, CH), jnp.int32),
        pltpu.VMEM((2, KS, CH, DH), jnp.float32),
        pltpu.SemaphoreType.DMA,
        pltpu.SemaphoreType.DMA,
    ],
)(_seg_body)


def _deg_finish_body(cnt_ref, dinv_ref):
    c = jnp.sum(cnt_ref[...], axis=0)
    dinv_ref[...] = lax.rsqrt(2.0 * c)


def _deg_finish(cnt):
    return pl.pallas_call(
        _deg_finish_body,
        out_shape=jax.ShapeDtypeStruct((CNT_WORDS // 128, 128), jnp.float32),
    )(cnt)


_RB = 1000


def _pre_body(ue_ref, ie_ref, vf_ref, tf_ref, su_ref, si_ref,
              wv_ref, bv_ref, wt_ref, bt_ref,
              vd_ref, td_ref,
              zua_ref, zub_ref, z0a_ref, z0b_ref,
              z1a_ref, z1b_ref, z2a_ref, z2b_ref):
    vf = vf_ref[...]
    tf = tf_ref[...]
    vn = vf * lax.rsqrt(jnp.maximum(jnp.sum(vf * vf, axis=1, keepdims=True),
                                    1e-24))
    tn = tf * lax.rsqrt(jnp.maximum(jnp.sum(tf * tf, axis=1, keepdims=True),
                                    1e-24))
    vd = lax.dot_general(vn, wv_ref[...], (((1,), (1,)), ((), ())),
                         preferred_element_type=jnp.float32) + bv_ref[...]
    td = lax.dot_general(tn, wt_ref[...], (((1,), (1,)), ((), ())),
                         preferred_element_type=jnp.float32) + bt_ref[...]
    su = su_ref[...]
    si = si_ref[...]
    zu = su * ue_ref[...]
    z0 = si * ie_ref[...]
    z1 = si * vd
    z2 = si * td
    vd_ref[...] = vd
    td_ref[...] = td
    zua_ref[...] = zu[:, :DH]
    zub_ref[...] = zu[:, DH:]
    z0a_ref[...] = z0[:, :DH]
    z0b_ref[...] = z0[:, DH:]
    z1a_ref[...] = z1[:, :DH]
    z1b_ref[...] = z1[:, DH:]
    z2a_ref[...] = z2[:, :DH]
    z2b_ref[...] = z2[:, DH:]


def _pre(user_emb, item_emb, v_feat, t_feat, s_u, s_i, Wv, bv, Wt, bt):
    grid = (NU // _RB,)
    rb = lambda i: (i, 0)
    full = lambda i: (0, 0)
    out_shapes = (jax.ShapeDtypeStruct((NU, D), jnp.float32),
                  jax.ShapeDtypeStruct((NU, D), jnp.float32)) + tuple(
        jax.ShapeDtypeStruct((NU, DH), jnp.float32) for _ in range(8))
    return pl.pallas_call(
        _pre_body,
        grid=grid,
        in_specs=[
            pl.BlockSpec((_RB, D), rb), pl.BlockSpec((_RB, D), rb),
            pl.BlockSpec((_RB, 128), rb), pl.BlockSpec((_RB, 128), rb),
            pl.BlockSpec((_RB, 1), rb), pl.BlockSpec((_RB, 1), rb),
            pl.BlockSpec((D, 128), full), pl.BlockSpec((1, D), full),
            pl.BlockSpec((D, 128), full), pl.BlockSpec((1, D), full),
        ],
        out_specs=(pl.BlockSpec((_RB, D), rb), pl.BlockSpec((_RB, D), rb))
        + tuple(pl.BlockSpec((_RB, DH), rb) for _ in range(8)),
        out_shape=out_shapes,
    )(user_emb, item_emb, v_feat, t_feat, s_u, s_i, Wv, bv, Wt, bt)


RP4 = ROWS_PAD // 4
_MB = 1568


def _mid_body(u0a_ref, u0b_ref, u1a_ref, u1b_ref, u2a_ref, u2b_ref,
              uia_ref, uib_ref, su_ref, si_ref,
              z0a_ref, z0b_ref, z1a_ref, z1b_ref, z2a_ref, z2b_ref,
              zia_ref, zib_ref):
    su2 = jnp.square(su_ref[...])
    si2 = jnp.square(si_ref[...])
    z0a_ref[...] = su2 * u0a_ref[...]
    z0b_ref[...] = su2 * u0b_ref[...]
    z1a_ref[...] = su2 * u1a_ref[...]
    z1b_ref[...] = su2 * u1b_ref[...]
    z2a_ref[...] = su2 * u2a_ref[...]
    z2b_ref[...] = su2 * u2b_ref[...]
    zia_ref[...] = si2 * uia_ref[...]
    zib_ref[...] = si2 * uib_ref[...]


def _mid(u0, u1, u2, ui, su_pk, si_pk):
    grid = (RP4 // _MB,)
    rb = lambda i: (i, 0)
    return pl.pallas_call(
        _mid_body,
        grid=grid,
        in_specs=[pl.BlockSpec((_MB, 4 * DH), rb)] * 10,
        out_specs=tuple(pl.BlockSpec((_MB, 4 * DH), rb) for _ in range(8)),
        out_shape=tuple(jax.ShapeDtypeStruct((RP4, 4 * DH), jnp.float32)
                        for _ in range(8)),
    )(*u0, *u1, *u2, *ui, su_pk, si_pk)


def _final_body(x0_ref, x1_ref, x2_ref,
                a0a_ref, a0b_ref, a1a_ref, a1b_ref, a2a_ref, a2b_ref,
                b0a_ref, b0b_ref, b1a_ref, b1b_ref, b2a_ref, b2b_ref,
                s_ref, w_ref, bias_ref, out_ref):
    s = s_ref[...]
    third = 1.0 / 3.0
    m0a = (x0_ref[...][:, :DH] + s * (a0a_ref[...] + b0a_ref[...])) * third
    m0b = (x0_ref[...][:, DH:] + s * (a0b_ref[...] + b0b_ref[...])) * third
    m1a = (x1_ref[...][:, :DH] + s * (a1a_ref[...] + b1a_ref[...])) * third
    m1b = (x1_ref[...][:, DH:] + s * (a1b_ref[...] + b1b_ref[...])) * third
    m2a = (x2_ref[...][:, :DH] + s * (a2a_ref[...] + b2a_ref[...])) * third
    m2b = (x2_ref[...][:, DH:] + s * (a2b_ref[...] + b2b_ref[...])) * third
    m = jnp.concatenate([m0a, m0b, m1a, m1b, m2a, m2b], axis=1)
    out_ref[...] = lax.dot_general(
        m, w_ref[...], (((1,), (1,)), ((), ())),
        preferred_element_type=jnp.float32) + bias_ref[...]


def _final(xs, u1s, u2s, s, W, b):
    grid = (NU // _RB,)
    rb = lambda i: (i, 0)
    full = lambda i: (0, 0)
    halves = u1s + u2s
    return pl.pallas_call(
        _final_body,
        grid=grid,
        in_specs=[pl.BlockSpec((_RB, D), rb)] * 3
        + [pl.BlockSpec((_RB, DH), rb)] * 12
        + [pl.BlockSpec((_RB, 1), rb),
           pl.BlockSpec((D, 3 * D), full), pl.BlockSpec((1, D), full)],
        out_specs=pl.BlockSpec((_RB, D), rb),
        out_shape=jax.ShapeDtypeStruct((NU, D), jnp.float32),
    )(*xs, *halves, s, W, b)


def _pipeline(user_emb, item_emb, v_feat, t_feat, Wv, bv, Wt, bt,
              Wu, bu, Wi, bi, edge_index):
    row = edge_index[0]
    col = edge_index[1]
    colL = col - NU
    bv = bv.reshape(1, D)
    bt = bt.reshape(1, D)
    bu = bu.reshape(1, D)
    bi = bi.reshape(1, D)

    pad_src = jnp.zeros((EPAD - E,), jnp.int32)
    pad_dst = jnp.full((EPAD - E,), ABSORB, jnp.int32)
    row_src = jnp.concatenate([row, pad_src]).reshape(NCROW, CH)
    row_dst = jnp.concatenate([row, pad_dst]).reshape(NCROW, CH)
    colL_src = jnp.concatenate([colL, pad_src]).reshape(NCROW, CH)
    colL_dst = jnp.concatenate([colL, pad_dst]).reshape(NCROW, CH)

    cnt = _deg_kernel(row, col)
    dinv = _deg_finish(cnt.reshape(NC * NS, CNT_WORDS // 128, 128))
    dflat = dinv.reshape(-1)
    s_u = dflat[:NU].reshape(NU, 1)
    s_i = dflat[NU:NN].reshape(NI, 1)
    su_pk = jnp.broadcast_to(dflat[:ROWS_PAD, None],
                             (ROWS_PAD, DH)).reshape(RP4, 4 * DH)
    si_pk = jnp.broadcast_to(
        lax.dynamic_slice(dflat, (NU,), (ROWS_PAD,))[:, None],
        (ROWS_PAD, DH)).reshape(RP4, 4 * DH)

    (v_dense, t_dense, zua, zub,
     z0a, z0b, z1a, z1b, z2a, z2b) = _pre(
        user_emb, item_emb, v_feat, t_feat, s_u, s_i, Wv, bv, Wt, bt)

    zeros = jnp.zeros((ROWS_PAD, DH), jnp.float32)

    def seg4(sd0, t0, sd1, t1, sd2, t2, sd3, t3):
        return _seg_kernel(
            zeros,
            sd0[0], sd0[1], t0[0], t0[1],
            sd1[0], sd1[1], t1[0], t1[1],
            sd2[0], sd2[1], t2[0], t2[1],
            sd3[0], sd3[1], t3[0], t3[1])

    iu = (colL_src, row_dst)
    ui = (row_src, colL_dst)

    (U1u0a, U1u0b, U1u1a, U1u1b,
     U1u2a, U1u2b, U1ia, U1ib) = seg4(
        iu, (z0a, z0b), iu, (z1a, z1b), iu, (z2a, z2b), ui, (zua, zub))

    pk = lambda a: a.reshape(RP4, 4 * DH)
    (Z1u0a, Z1u0b, Z1u1a, Z1u1b,
     Z1u2a, Z1u2b, Z1ia, Z1ib) = map(lambda a: a.reshape(ROWS_PAD, DH), _mid(
        (pk(U1u0a), pk(U1u0b)), (pk(U1u1a), pk(U1u1b)),
        (pk(U1u2a), pk(U1u2b)), (pk(U1ia), pk(U1ib)),
        su_pk, si_pk))

    (U2ua, U2ub, U2i0a, U2i0b,
     U2i1a, U2i1b, U2i2a, U2i2b) = seg4(
        iu, (Z1ia, Z1ib), ui, (Z1u0a, Z1u0b),
        ui, (Z1u1a, Z1u1b), ui, (Z1u2a, Z1u2b))

    user = _final((user_emb, user_emb, user_emb),
                  (U1u0a, U1u0b, U1u1a, U1u1b, U1u2a, U1u2b),
                  (U2ua, U2ub, U2ua, U2ub, U2ua, U2ub), s_u, Wu, bu)
    item = _final((item_emb, v_dense, t_dense),
                  (U1ia, U1ib, U1ia, U1ib, U1ia, U1ib),
                  (U2i0a, U2i0b, U2i1a, U2i1b, U2i2a, U2i2b), s_i, Wi, bi)
    return (user, item)


def kernel(user_emb, item_emb, v_feat, t_feat, Wv, bv, Wt, bt,
           Wu, bu, Wi, bi, edge_index):
    return _pipeline(user_emb, item_emb, v_feat, t_feat, Wv, bv, Wt, bt,
                     Wu, bu, Wi, bi, edge_index)


# speedup vs baseline: 1.1995x; 1.0161x over previous
import functools

import jax
import jax.numpy as jnp
from jax import lax
from jax.experimental import pallas as pl
from jax.experimental.pallas import tpu as pltpu
from jax.experimental.pallas import tpu_sc as plsc

NU = 25000
NI = 25000
NN = NU + NI
D = 64
E = 800000

NC = 2
NS = 16
CH = 128
KS = 8
NSUPER = 50
NJ = NSUPER // 2
NCHUNK = NSUPER * KS
EPT = NCHUNK * CH
EPAD = EPT * NS
NCROW = EPAD // CH
DH = 32
ROWS_PAD = 25088
STRIPE = ROWS_PAD // NS
ABSORB = 25080

EPW = E // (NC * NS)
CNT_WORDS = 51200

_MESH = plsc.VectorSubcoreMesh(
    core_axis_name="c", subcore_axis_name="s", num_cores=NC, num_subcores=NS)


def _wid():
    return lax.axis_index("s") * NC + lax.axis_index("c")


def _deg_body(row_hbm, col_hbm, out_hbm, cnt, idxbuf):
    wid = _wid()
    zeros16 = jnp.zeros((16,), jnp.float32)
    ones16 = jnp.ones((16,), jnp.float32)

    def zero_body(i, c):
        cnt[pl.ds(i * 16, 16)] = zeros16
        return c
    lax.fori_loop(0, CNT_WORDS // 16, zero_body, 0)

    pltpu.sync_copy(row_hbm.at[pl.ds(wid * EPW, EPW)], idxbuf.at[pl.ds(0, EPW)])
    pltpu.sync_copy(col_hbm.at[pl.ds(wid * EPW, EPW)],
                    idxbuf.at[pl.ds(EPW, EPW)])

    def body(i, c):
        iv = idxbuf[pl.ds(i * 16, 16)]
        plsc.addupdate_scatter(cnt, [iv], ones16)
        return c
    lax.fori_loop(0, 2 * EPW // 16, body, 0)

    pltpu.sync_copy(cnt, out_hbm.at[wid])


_deg_kernel = functools.partial(
    pl.kernel,
    out_type=jax.ShapeDtypeStruct((NC * NS, CNT_WORDS), jnp.float32),
    mesh=_MESH,
    compiler_params=pltpu.CompilerParams(needs_layout_passes=False),
    scratch_types=[
        pltpu.VMEM((CNT_WORDS,), jnp.float32),
        pltpu.VMEM((2 * EPW,), jnp.int32),
    ],
)(_deg_body)


def _seg_body(zeros_hbm,
              s0, d0, t0a, t0b, s1, d1, t1a, t1b,
              s2, d2, t2a, t2b, s3, d3, t3a, t3b,
              o0a, o0b, o1a, o1b, o2a, o2b, o3a, o3b,
              acc, sidx, didx, rows, gsem, ssem):
    c = lax.axis_index("c")
    sid = lax.axis_index("s")
    r0 = sid * STRIPE

    def run(src, dst, tab, out):
        crow = sid * NCHUNK

        def load_idx(sup, p):
            pltpu.sync_copy(src.at[pl.ds(crow + sup * KS, KS)], sidx.at[p])
            pltpu.sync_copy(dst.at[pl.ds(crow + sup * KS, KS)], didx.at[p])

        def fire_gathers(p):
            for k in range(KS):
                pltpu.async_copy(tab.at[sidx.at[p, k]], rows.at[p, k], gsem)

        def drain_gathers(p):
            for k in range(KS):
                pltpu.make_async_copy(tab.at[sidx.at[p, k]],
                                      rows.at[p, k], gsem).wait()

        def fire_scatters(p):
            for k in range(KS):
                pltpu.async_copy(rows.at[p, k], acc.at[didx.at[p, k]],
                                 ssem, add=True)

        def drain_scatters(p):
            for k in range(KS):
                pltpu.make_async_copy(rows.at[p, k],
                                      acc.at[didx.at[p, k]], ssem).wait()

        load_idx(0, 0)
        fire_gathers(0)
        pltpu.sync_copy(zeros_hbm.at[pl.ds(r0, STRIPE)],
                        acc.at[pl.ds(r0, STRIPE)])
        plsc.subcore_barrier()

        def body(j, carry):
            @pl.when(j > 0)
            def _():
                drain_scatters(1)
            load_idx(2 * j + 1, 1)
            fire_gathers(1)
            drain_gathers(0)
            fire_scatters(0)
            drain_scatters(0)

            @pl.when(j < NJ - 1)
            def _():
                load_idx(2 * j + 2, 0)
                fire_gathers(0)
            drain_gathers(1)
            fire_scatters(1)
            return carry
        lax.fori_loop(0, NJ, body, 0)
        drain_scatters(1)
        plsc.subcore_barrier()
        pltpu.sync_copy(acc.at[pl.ds(r0, STRIPE)], out.at[pl.ds(r0, STRIPE)])

    @pl.when(c == 0)
    def _():
        run(s0, d0, t0a, o0a)
        run(s0, d0, t0b, o0b)
        run(s1, d1, t1a, o1a)
        run(s1, d1, t1b, o1b)

    @pl.when(c == 1)
    def _():
        run(s2, d2, t2a, o2a)
        run(s2, d2, t2b, o2b)
        run(s3, d3, t3a, o3a)
        run(s3, d3, t3b, o3b)


_OUT8 = tuple(jax.ShapeDtypeStruct((ROWS_PAD, DH), jnp.float32)
              for _ in range(8))

_seg_kernel = functools.partial(
    pl.kernel,
    out_type=_OUT8,
    mesh=_MESH,
    compiler_params=pltpu.CompilerParams(use_tc_tiling_on_sc=False),
    scratch_types=[
        pltpu.MemorySpace.VMEM_SHARED((ROWS_PAD, DH), jnp.float32),
        pltpu.VMEM((2, KS, CH), jnp.int32),
        pltpu.VMEM((2, KS, CH), jnp.int32),
        pltpu.VMEM((2, KS, CH, DH), jnp.float32),
        pltpu.SemaphoreType.DMA,
        pltpu.SemaphoreType.DMA,
    ],
)(_seg_body)


def _deg_finish_body(cnt_ref, dinv_ref):
    c = jnp.sum(cnt_ref[...], axis=0)
    dinv_ref[...] = lax.rsqrt(2.0 * c)


def _deg_finish(cnt):
    return pl.pallas_call(
        _deg_finish_body,
        out_shape=jax.ShapeDtypeStruct((CNT_WORDS // 128, 128), jnp.float32),
    )(cnt)


_RB = 1000


def _pre_body(ue_ref, ie_ref, vf_ref, tf_ref, su_ref, si_ref,
              wv_ref, bv_ref, wt_ref, bt_ref,
              vd_ref, td_ref,
              zua_ref, zub_ref, z0a_ref, z0b_ref,
              z1a_ref, z1b_ref, z2a_ref, z2b_ref):
    vf = vf_ref[...]
    tf = tf_ref[...]
    vn = vf * lax.rsqrt(jnp.maximum(jnp.sum(vf * vf, axis=1, keepdims=True),
                                    1e-24))
    tn = tf * lax.rsqrt(jnp.maximum(jnp.sum(tf * tf, axis=1, keepdims=True),
                                    1e-24))
    vd = lax.dot_general(vn, wv_ref[...], (((1,), (1,)), ((), ())),
                         preferred_element_type=jnp.float32) + bv_ref[...]
    td = lax.dot_general(tn, wt_ref[...], (((1,), (1,)), ((), ())),
                         preferred_element_type=jnp.float32) + bt_ref[...]
    su = su_ref[...]
    si = si_ref[...]
    zu = su * ue_ref[...]
    z0 = si * ie_ref[...]
    z1 = si * vd
    z2 = si * td
    vd_ref[...] = vd
    td_ref[...] = td
    zua_ref[...] = zu[:, :DH]
    zub_ref[...] = zu[:, DH:]
    z0a_ref[...] = z0[:, :DH]
    z0b_ref[...] = z0[:, DH:]
    z1a_ref[...] = z1[:, :DH]
    z1b_ref[...] = z1[:, DH:]
    z2a_ref[...] = z2[:, :DH]
    z2b_ref[...] = z2[:, DH:]


def _pre(user_emb, item_emb, v_feat, t_feat, s_u, s_i, Wv, bv, Wt, bt):
    grid = (NU // _RB,)
    rb = lambda i: (i, 0)
    full = lambda i: (0, 0)
    out_shapes = (jax.ShapeDtypeStruct((NU, D), jnp.float32),
                  jax.ShapeDtypeStruct((NU, D), jnp.float32)) + tuple(
        jax.ShapeDtypeStruct((NU, DH), jnp.float32) for _ in range(8))
    return pl.pallas_call(
        _pre_body,
        grid=grid,
        in_specs=[
            pl.BlockSpec((_RB, D), rb), pl.BlockSpec((_RB, D), rb),
            pl.BlockSpec((_RB, 128), rb), pl.BlockSpec((_RB, 128), rb),
            pl.BlockSpec((_RB, 1), rb), pl.BlockSpec((_RB, 1), rb),
            pl.BlockSpec((D, 128), full), pl.BlockSpec((1, D), full),
            pl.BlockSpec((D, 128), full), pl.BlockSpec((1, D), full),
        ],
        out_specs=(pl.BlockSpec((_RB, D), rb), pl.BlockSpec((_RB, D), rb))
        + tuple(pl.BlockSpec((_RB, DH), rb) for _ in range(8)),
        out_shape=out_shapes,
    )(user_emb, item_emb, v_feat, t_feat, s_u, s_i, Wv, bv, Wt, bt)


RP4 = ROWS_PAD // 4
_MB = 1568


def _mid_body(u0a_ref, u0b_ref, u1a_ref, u1b_ref, u2a_ref, u2b_ref,
              uia_ref, uib_ref, su_ref, si_ref,
              z0a_ref, z0b_ref, z1a_ref, z1b_ref, z2a_ref, z2b_ref,
              zia_ref, zib_ref):
    su2 = jnp.square(su_ref[...])
    si2 = jnp.square(si_ref[...])
    z0a_ref[...] = su2 * u0a_ref[...]
    z0b_ref[...] = su2 * u0b_ref[...]
    z1a_ref[...] = su2 * u1a_ref[...]
    z1b_ref[...] = su2 * u1b_ref[...]
    z2a_ref[...] = su2 * u2a_ref[...]
    z2b_ref[...] = su2 * u2b_ref[...]
    zia_ref[...] = si2 * uia_ref[...]
    zib_ref[...] = si2 * uib_ref[...]


def _mid(u0, u1, u2, ui, su_pk, si_pk):
    grid = (RP4 // _MB,)
    rb = lambda i: (i, 0)
    return pl.pallas_call(
        _mid_body,
        grid=grid,
        in_specs=[pl.BlockSpec((_MB, 4 * DH), rb)] * 10,
        out_specs=tuple(pl.BlockSpec((_MB, 4 * DH), rb) for _ in range(8)),
        out_shape=tuple(jax.ShapeDtypeStruct((RP4, 4 * DH), jnp.float32)
                        for _ in range(8)),
    )(*u0, *u1, *u2, *ui, su_pk, si_pk)


_HB = 784


def _head_pk_body(u1u0a, u1u0b, u1u1a, u1u1b, u1u2a, u1u2b, u2ua, u2ub,
                  u1ia, u1ib, u2i0a, u2i0b, u2i1a, u2i1b, u2i2a, u2i2b,
                  su, si, w2u, w2i, outu, outi):
    third = 1.0 / 3.0
    s = su[...] * third
    mu = jnp.concatenate([
        s * (u1u0a[...] + u2ua[...]), s * (u1u0b[...] + u2ub[...]),
        s * (u1u1a[...] + u2ua[...]), s * (u1u1b[...] + u2ub[...]),
        s * (u1u2a[...] + u2ua[...]), s * (u1u2b[...] + u2ub[...]),
    ], axis=1)
    outu[...] = lax.dot_general(mu, w2u[...], (((1,), (0,)), ((), ())),
                                preferred_element_type=jnp.float32)
    t = si[...] * third
    mi = jnp.concatenate([
        t * (u1ia[...] + u2i0a[...]), t * (u1ib[...] + u2i0b[...]),
        t * (u1ia[...] + u2i1a[...]), t * (u1ib[...] + u2i1b[...]),
        t * (u1ia[...] + u2i2a[...]), t * (u1ib[...] + u2i2b[...]),
    ], axis=1)
    outi[...] = lax.dot_general(mi, w2i[...], (((1,), (0,)), ((), ())),
                                preferred_element_type=jnp.float32)


def _head_pk(uhalves, ihalves, su_pk, si_pk, W2u, W2i):
    grid = (RP4 // _HB,)
    rb = lambda i: (i, 0)
    full = lambda i: (0, 0)
    return pl.pallas_call(
        _head_pk_body,
        grid=grid,
        in_specs=[pl.BlockSpec((_HB, 4 * DH), rb)] * 18
        + [pl.BlockSpec((6 * 4 * DH, 4 * D), full)] * 2,
        out_specs=(pl.BlockSpec((_HB, 4 * D), rb),) * 2,
        out_shape=(jax.ShapeDtypeStruct((RP4, 4 * D), jnp.float32),) * 2,
    )(*uhalves, *ihalves, su_pk, si_pk, W2u, W2i)


def _head_fin_body(x0, x1, x2, up, w, b, out):
    m = jnp.concatenate([x0[...], x1[...], x2[...]], axis=1)
    out[...] = lax.dot_general(
        m, w[...], (((1,), (1,)), ((), ())),
        preferred_element_type=jnp.float32) * (1.0 / 3.0) + up[...] + b[...]


def _head_fin(xs, up, W, b):
    grid = (NU // _RB,)
    rb = lambda i: (i, 0)
    full = lambda i: (0, 0)
    return pl.pallas_call(
        _head_fin_body,
        grid=grid,
        in_specs=[pl.BlockSpec((_RB, D), rb)] * 4
        + [pl.BlockSpec((D, 3 * D), full), pl.BlockSpec((1, D), full)],
        out_specs=pl.BlockSpec((_RB, D), rb),
        out_shape=jax.ShapeDtypeStruct((NU, D), jnp.float32),
    )(*xs, up, W, b)


def _w2(W):
    WT6 = W.T.reshape(6, DH, D)
    eye4 = jnp.eye(4, dtype=W.dtype)
    A = WT6[:, None, :, None, :] * eye4[None, :, None, :, None]
    return A.reshape(6 * 4 * DH, 4 * D)


def _pipeline(user_emb, item_emb, v_feat, t_feat, Wv, bv, Wt, bt,
              Wu, bu, Wi, bi, edge_index):
    row = edge_index[0]
    col = edge_index[1]
    colL = col - NU
    bv = bv.reshape(1, D)
    bt = bt.reshape(1, D)
    bu = bu.reshape(1, D)
    bi = bi.reshape(1, D)

    pad_src = jnp.zeros((EPAD - E,), jnp.int32)
    pad_dst = jnp.full((EPAD - E,), ABSORB, jnp.int32)
    row_src = jnp.concatenate([row, pad_src]).reshape(NCROW, CH)
    row_dst = jnp.concatenate([row, pad_dst]).reshape(NCROW, CH)
    colL_src = jnp.concatenate([colL, pad_src]).reshape(NCROW, CH)
    colL_dst = jnp.concatenate([colL, pad_dst]).reshape(NCROW, CH)

    cnt = _deg_kernel(row, col)
    dinv = _deg_finish(cnt.reshape(NC * NS, CNT_WORDS // 128, 128))
    dflat = dinv.reshape(-1)
    s_u = dflat[:NU].reshape(NU, 1)
    s_i = dflat[NU:NN].reshape(NI, 1)
    su_pk = jnp.broadcast_to(dflat[:ROWS_PAD, None],
                             (ROWS_PAD, DH)).reshape(RP4, 4 * DH)
    si_pk = jnp.broadcast_to(
        lax.dynamic_slice(dflat, (NU,), (ROWS_PAD,))[:, None],
        (ROWS_PAD, DH)).reshape(RP4, 4 * DH)

    (v_dense, t_dense, zua, zub,
     z0a, z0b, z1a, z1b, z2a, z2b) = _pre(
        user_emb, item_emb, v_feat, t_feat, s_u, s_i, Wv, bv, Wt, bt)

    zeros = jnp.zeros((ROWS_PAD, DH), jnp.float32)

    def seg4(sd0, t0, sd1, t1, sd2, t2, sd3, t3):
        return _seg_kernel(
            zeros,
            sd0[0], sd0[1], t0[0], t0[1],
            sd1[0], sd1[1], t1[0], t1[1],
            sd2[0], sd2[1], t2[0], t2[1],
            sd3[0], sd3[1], t3[0], t3[1])

    iu = (colL_src, row_dst)
    ui = (row_src, colL_dst)

    (U1u0a, U1u0b, U1u1a, U1u1b,
     U1u2a, U1u2b, U1ia, U1ib) = seg4(
        iu, (z0a, z0b), iu, (z1a, z1b), iu, (z2a, z2b), ui, (zua, zub))

    pk = lambda a: a.reshape(RP4, 4 * DH)
    (Z1u0a, Z1u0b, Z1u1a, Z1u1b,
     Z1u2a, Z1u2b, Z1ia, Z1ib) = map(lambda a: a.reshape(ROWS_PAD, DH), _mid(
        (pk(U1u0a), pk(U1u0b)), (pk(U1u1a), pk(U1u1b)),
        (pk(U1u2a), pk(U1u2b)), (pk(U1ia), pk(U1ib)),
        su_pk, si_pk))

    (U2ua, U2ub, U2i0a, U2i0b,
     U2i1a, U2i1b, U2i2a, U2i2b) = seg4(
        iu, (Z1ia, Z1ib), ui, (Z1u0a, Z1u0b),
        ui, (Z1u1a, Z1u1b), ui, (Z1u2a, Z1u2b))

    upu, upi = _head_pk(
        (pk(U1u0a), pk(U1u0b), pk(U1u1a), pk(U1u1b),
         pk(U1u2a), pk(U1u2b), pk(U2ua), pk(U2ub)),
        (pk(U1ia), pk(U1ib), pk(U2i0a), pk(U2i0b),
         pk(U2i1a), pk(U2i1b), pk(U2i2a), pk(U2i2b)),
        su_pk, si_pk, _w2(Wu), _w2(Wi))
    user = _head_fin((user_emb, user_emb, user_emb),
                     upu.reshape(ROWS_PAD, D), Wu, bu)
    item = _head_fin((item_emb, v_dense, t_dense),
                     upi.reshape(ROWS_PAD, D), Wi, bi)
    return (user, item)


def kernel(user_emb, item_emb, v_feat, t_feat, Wv, bv, Wt, bt,
           Wu, bu, Wi, bi, edge_index):
    return _pipeline(user_emb, item_emb, v_feat, t_feat, Wv, bv, Wt, bt,
                     Wu, bu, Wi, bi, edge_index)


# speedup vs baseline: 1.2013x; 1.0015x over previous
import functools

import jax
import jax.numpy as jnp
from jax import lax
from jax.experimental import pallas as pl
from jax.experimental.pallas import tpu as pltpu
from jax.experimental.pallas import tpu_sc as plsc

NU = 25000
NI = 25000
NN = NU + NI
D = 64
E = 800000

NC = 2
NS = 16
CH = 128
KS = 8
NSUPER = 50
NJ = NSUPER // 2
NCHUNK = NSUPER * KS
EPT = NCHUNK * CH
EPAD = EPT * NS
NCROW = EPAD // CH
DH = 32
ROWS_PAD = 25088
STRIPE = ROWS_PAD // NS
ABSORB = 25080

EPW = E // (NC * NS)
CNT_WORDS = 51200

_MESH = plsc.VectorSubcoreMesh(
    core_axis_name="c", subcore_axis_name="s", num_cores=NC, num_subcores=NS)


def _wid():
    return lax.axis_index("s") * NC + lax.axis_index("c")


def _deg_body(row_hbm, col_hbm, out_hbm, cnt, idxbuf):
    wid = _wid()
    zeros16 = jnp.zeros((16,), jnp.float32)
    ones16 = jnp.ones((16,), jnp.float32)

    def zero_body(i, c):
        cnt[pl.ds(i * 16, 16)] = zeros16
        return c
    lax.fori_loop(0, CNT_WORDS // 16, zero_body, 0)

    pltpu.sync_copy(row_hbm.at[pl.ds(wid * EPW, EPW)], idxbuf.at[pl.ds(0, EPW)])
    pltpu.sync_copy(col_hbm.at[pl.ds(wid * EPW, EPW)],
                    idxbuf.at[pl.ds(EPW, EPW)])

    def body(i, c):
        iv = idxbuf[pl.ds(i * 16, 16)]
        plsc.addupdate_scatter(cnt, [iv], ones16)
        return c
    lax.fori_loop(0, 2 * EPW // 16, body, 0)

    pltpu.sync_copy(cnt, out_hbm.at[wid])


_deg_kernel = functools.partial(
    pl.kernel,
    out_type=jax.ShapeDtypeStruct((NC * NS, CNT_WORDS), jnp.float32),
    mesh=_MESH,
    compiler_params=pltpu.CompilerParams(needs_layout_passes=False),
    scratch_types=[
        pltpu.VMEM((CNT_WORDS,), jnp.float32),
        pltpu.VMEM((2 * EPW,), jnp.int32),
    ],
)(_deg_body)


def _seg_body(zeros_hbm,
              s0, d0, t0a, t0b, s1, d1, t1a, t1b,
              s2, d2, t2a, t2b, s3, d3, t3a, t3b,
              o0a, o0b, o1a, o1b, o2a, o2b, o3a, o3b,
              acc, sidx, didx, rows, gsem, ssem):
    c = lax.axis_index("c")
    sid = lax.axis_index("s")
    r0 = sid * STRIPE

    def run(src, dst, tab, out):
        crow = sid * NCHUNK

        def load_idx(sup, p):
            pltpu.sync_copy(src.at[pl.ds(crow + sup * KS, KS)], sidx.at[p])
            pltpu.sync_copy(dst.at[pl.ds(crow + sup * KS, KS)], didx.at[p])

        def fire_gathers(p):
            for k in range(KS):
                pltpu.async_copy(tab.at[sidx.at[p, k]], rows.at[p, k], gsem)

        def drain_gathers(p):
            for k in range(KS):
                pltpu.make_async_copy(tab.at[sidx.at[p, k]],
                                      rows.at[p, k], gsem).wait()

        def fire_scatters(p):
            for k in range(KS):
                pltpu.async_copy(rows.at[p, k], acc.at[didx.at[p, k]],
                                 ssem, add=True)

        def drain_scatters(p):
            for k in range(KS):
                pltpu.make_async_copy(rows.at[p, k],
                                      acc.at[didx.at[p, k]], ssem).wait()

        load_idx(0, 0)
        fire_gathers(0)
        pltpu.sync_copy(zeros_hbm.at[pl.ds(r0, STRIPE)],
                        acc.at[pl.ds(r0, STRIPE)])
        plsc.subcore_barrier()

        def body(j, carry):
            @pl.when(j > 0)
            def _():
                drain_scatters(1)
            load_idx(2 * j + 1, 1)
            fire_gathers(1)
            drain_gathers(0)
            fire_scatters(0)
            drain_scatters(0)

            @pl.when(j < NJ - 1)
            def _():
                load_idx(2 * j + 2, 0)
                fire_gathers(0)
            drain_gathers(1)
            fire_scatters(1)
            return carry
        lax.fori_loop(0, NJ, body, 0)
        drain_scatters(1)
        plsc.subcore_barrier()
        pltpu.sync_copy(acc.at[pl.ds(r0, STRIPE)], out.at[pl.ds(r0, STRIPE)])

    @pl.when(c == 0)
    def _():
        run(s0, d0, t0a, o0a)
        run(s0, d0, t0b, o0b)
        run(s1, d1, t1a, o1a)
        run(s1, d1, t1b, o1b)

    @pl.when(c == 1)
    def _():
        run(s2, d2, t2a, o2a)
        run(s2, d2, t2b, o2b)
        run(s3, d3, t3a, o3a)
        run(s3, d3, t3b, o3b)


_OUT8 = tuple(jax.ShapeDtypeStruct((ROWS_PAD, DH), jnp.float32)
              for _ in range(8))

_seg_kernel = functools.partial(
    pl.kernel,
    out_type=_OUT8,
    mesh=_MESH,
    compiler_params=pltpu.CompilerParams(use_tc_tiling_on_sc=False),
    scratch_types=[
        pltpu.MemorySpace.VMEM_SHARED((ROWS_PAD, DH), jnp.float32),
        pltpu.VMEM((2, KS, CH), jnp.int32),
        pltpu.VMEM((2, KS, CH), jnp.int32),
        pltpu.VMEM((2, KS, CH, DH), jnp.float32),
        pltpu.SemaphoreType.DMA,
        pltpu.SemaphoreType.DMA,
    ],
)(_seg_body)


def _deg_finish_body(cnt_ref, dinv_ref):
    c = jnp.sum(cnt_ref[...], axis=0)
    dinv_ref[...] = lax.rsqrt(2.0 * c)


def _deg_finish(cnt):
    return pl.pallas_call(
        _deg_finish_body,
        out_shape=jax.ShapeDtypeStruct((CNT_WORDS // 128, 128), jnp.float32),
    )(cnt)


_RB = 1000


def _proj_body(vf_ref, tf_ref, wv_ref, bv_ref, wt_ref, bt_ref,
               vd_ref, td_ref):
    vf = vf_ref[...]
    tf = tf_ref[...]
    vn = vf * lax.rsqrt(jnp.maximum(jnp.sum(vf * vf, axis=1, keepdims=True),
                                    1e-24))
    tn = tf * lax.rsqrt(jnp.maximum(jnp.sum(tf * tf, axis=1, keepdims=True),
                                    1e-24))
    vd_ref[...] = lax.dot_general(
        vn, wv_ref[...], (((1,), (1,)), ((), ())),
        preferred_element_type=jnp.float32) + bv_ref[...]
    td_ref[...] = lax.dot_general(
        tn, wt_ref[...], (((1,), (1,)), ((), ())),
        preferred_element_type=jnp.float32) + bt_ref[...]


def _proj(v_feat, t_feat, Wv, bv, Wt, bt):
    grid = (NU // _RB,)
    rb = lambda i: (i, 0)
    full = lambda i: (0, 0)
    return pl.pallas_call(
        _proj_body,
        grid=grid,
        in_specs=[
            pl.BlockSpec((_RB, 128), rb), pl.BlockSpec((_RB, 128), rb),
            pl.BlockSpec((D, 128), full), pl.BlockSpec((1, D), full),
            pl.BlockSpec((D, 128), full), pl.BlockSpec((1, D), full),
        ],
        out_specs=(pl.BlockSpec((_RB, D), rb), pl.BlockSpec((_RB, D), rb)),
        out_shape=(jax.ShapeDtypeStruct((NU, D), jnp.float32),
                   jax.ShapeDtypeStruct((NU, D), jnp.float32)),
    )(v_feat, t_feat, Wv, bv, Wt, bt)


def _pre_body(ue_ref, ie_ref, vd_in_ref, td_in_ref, su_ref, si_ref,
              zua_ref, zub_ref, z0a_ref, z0b_ref,
              z1a_ref, z1b_ref, z2a_ref, z2b_ref):
    su = su_ref[...]
    si = si_ref[...]
    zu = su * ue_ref[...]
    z0 = si * ie_ref[...]
    z1 = si * vd_in_ref[...]
    z2 = si * td_in_ref[...]
    zua_ref[...] = zu[:, :DH]
    zub_ref[...] = zu[:, DH:]
    z0a_ref[...] = z0[:, :DH]
    z0b_ref[...] = z0[:, DH:]
    z1a_ref[...] = z1[:, :DH]
    z1b_ref[...] = z1[:, DH:]
    z2a_ref[...] = z2[:, :DH]
    z2b_ref[...] = z2[:, DH:]


def _pre(user_emb, item_emb, v_dense, t_dense, s_u, s_i):
    grid = (NU // _RB,)
    rb = lambda i: (i, 0)
    return pl.pallas_call(
        _pre_body,
        grid=grid,
        in_specs=[pl.BlockSpec((_RB, D), rb)] * 4
        + [pl.BlockSpec((_RB, 1), rb)] * 2,
        out_specs=tuple(pl.BlockSpec((_RB, DH), rb) for _ in range(8)),
        out_shape=tuple(jax.ShapeDtypeStruct((NU, DH), jnp.float32)
                        for _ in range(8)),
    )(user_emb, item_emb, v_dense, t_dense, s_u, s_i)


RP4 = ROWS_PAD // 4
_MB = 1568


def _mid_body(u0a_ref, u0b_ref, u1a_ref, u1b_ref, u2a_ref, u2b_ref,
              uia_ref, uib_ref, su_ref, si_ref,
              z0a_ref, z0b_ref, z1a_ref, z1b_ref, z2a_ref, z2b_ref,
              zia_ref, zib_ref):
    su2 = jnp.square(su_ref[...])
    si2 = jnp.square(si_ref[...])
    z0a_ref[...] = su2 * u0a_ref[...]
    z0b_ref[...] = su2 * u0b_ref[...]
    z1a_ref[...] = su2 * u1a_ref[...]
    z1b_ref[...] = su2 * u1b_ref[...]
    z2a_ref[...] = su2 * u2a_ref[...]
    z2b_ref[...] = su2 * u2b_ref[...]
    zia_ref[...] = si2 * uia_ref[...]
    zib_ref[...] = si2 * uib_ref[...]


def _mid(u0, u1, u2, ui, su_pk, si_pk):
    grid = (RP4 // _MB,)
    rb = lambda i: (i, 0)
    return pl.pallas_call(
        _mid_body,
        grid=grid,
        in_specs=[pl.BlockSpec((_MB, 4 * DH), rb)] * 10,
        out_specs=tuple(pl.BlockSpec((_MB, 4 * DH), rb) for _ in range(8)),
        out_shape=tuple(jax.ShapeDtypeStruct((RP4, 4 * DH), jnp.float32)
                        for _ in range(8)),
    )(*u0, *u1, *u2, *ui, su_pk, si_pk)


_HB = 784


def _head_pk_body(u1u0a, u1u0b, u1u1a, u1u1b, u1u2a, u1u2b, u2ua, u2ub,
                  u1ia, u1ib, u2i0a, u2i0b, u2i1a, u2i1b, u2i2a, u2i2b,
                  su, si, w2u, w2i, outu, outi):
    third = 1.0 / 3.0
    s = su[...] * third
    mu = jnp.concatenate([
        s * (u1u0a[...] + u2ua[...]), s * (u1u0b[...] + u2ub[...]),
        s * (u1u1a[...] + u2ua[...]), s * (u1u1b[...] + u2ub[...]),
        s * (u1u2a[...] + u2ua[...]), s * (u1u2b[...] + u2ub[...]),
    ], axis=1)
    outu[...] = lax.dot_general(mu, w2u[...], (((1,), (0,)), ((), ())),
                                preferred_element_type=jnp.float32)
    t = si[...] * third
    mi = jnp.concatenate([
        t * (u1ia[...] + u2i0a[...]), t * (u1ib[...] + u2i0b[...]),
        t * (u1ia[...] + u2i1a[...]), t * (u1ib[...] + u2i1b[...]),
        t * (u1ia[...] + u2i2a[...]), t * (u1ib[...] + u2i2b[...]),
    ], axis=1)
    outi[...] = lax.dot_general(mi, w2i[...], (((1,), (0,)), ((), ())),
                                preferred_element_type=jnp.float32)


def _head_pk(uhalves, ihalves, su_pk, si_pk, W2u, W2i):
    grid = (RP4 // _HB,)
    rb = lambda i: (i, 0)
    full = lambda i: (0, 0)
    return pl.pallas_call(
        _head_pk_body,
        grid=grid,
        in_specs=[pl.BlockSpec((_HB, 4 * DH), rb)] * 18
        + [pl.BlockSpec((6 * 4 * DH, 4 * D), full)] * 2,
        out_specs=(pl.BlockSpec((_HB, 4 * D), rb),) * 2,
        out_shape=(jax.ShapeDtypeStruct((RP4, 4 * D), jnp.float32),) * 2,
    )(*uhalves, *ihalves, su_pk, si_pk, W2u, W2i)


def _head_fin_body(x0, x1, x2, up, w, b, out):
    m = jnp.concatenate([x0[...], x1[...], x2[...]], axis=1)
    out[...] = lax.dot_general(
        m, w[...], (((1,), (1,)), ((), ())),
        preferred_element_type=jnp.float32) * (1.0 / 3.0) + up[...] + b[...]


def _head_fin(xs, up, W, b):
    grid = (NU // _RB,)
    rb = lambda i: (i, 0)
    full = lambda i: (0, 0)
    return pl.pallas_call(
        _head_fin_body,
        grid=grid,
        in_specs=[pl.BlockSpec((_RB, D), rb)] * 4
        + [pl.BlockSpec((D, 3 * D), full), pl.BlockSpec((1, D), full)],
        out_specs=pl.BlockSpec((_RB, D), rb),
        out_shape=jax.ShapeDtypeStruct((NU, D), jnp.float32),
    )(*xs, up, W, b)


def _w2(W):
    WT6 = W.T.reshape(6, DH, D)
    eye4 = jnp.eye(4, dtype=W.dtype)
    A = WT6[:, None, :, None, :] * eye4[None, :, None, :, None]
    return A.reshape(6 * 4 * DH, 4 * D)


def _pipeline(user_emb, item_emb, v_feat, t_feat, Wv, bv, Wt, bt,
              Wu, bu, Wi, bi, edge_index):
    row = edge_index[0]
    col = edge_index[1]
    colL = col - NU
    bv = bv.reshape(1, D)
    bt = bt.reshape(1, D)
    bu = bu.reshape(1, D)
    bi = bi.reshape(1, D)

    pad_src = jnp.zeros((EPAD - E,), jnp.int32)
    pad_dst = jnp.full((EPAD - E,), ABSORB, jnp.int32)
    row_src = jnp.concatenate([row, pad_src]).reshape(NCROW, CH)
    row_dst = jnp.concatenate([row, pad_dst]).reshape(NCROW, CH)
    colL_src = jnp.concatenate([colL, pad_src]).reshape(NCROW, CH)
    colL_dst = jnp.concatenate([colL, pad_dst]).reshape(NCROW, CH)

    v_dense, t_dense = _proj(v_feat, t_feat, Wv, bv, Wt, bt)

    cnt = _deg_kernel(row, col)
    dinv = _deg_finish(cnt.reshape(NC * NS, CNT_WORDS // 128, 128))
    dflat = dinv.reshape(-1)
    s_u = dflat[:NU].reshape(NU, 1)
    s_i = dflat[NU:NN].reshape(NI, 1)
    su_pk = jnp.broadcast_to(dflat[:ROWS_PAD, None],
                             (ROWS_PAD, DH)).reshape(RP4, 4 * DH)
    si_pk = jnp.broadcast_to(
        lax.dynamic_slice(dflat, (NU,), (ROWS_PAD,))[:, None],
        (ROWS_PAD, DH)).reshape(RP4, 4 * DH)

    (zua, zub, z0a, z0b, z1a, z1b, z2a, z2b) = _pre(
        user_emb, item_emb, v_dense, t_dense, s_u, s_i)

    zeros = jnp.zeros((ROWS_PAD, DH), jnp.float32)

    def seg4(sd0, t0, sd1, t1, sd2, t2, sd3, t3):
        return _seg_kernel(
            zeros,
            sd0[0], sd0[1], t0[0], t0[1],
            sd1[0], sd1[1], t1[0], t1[1],
            sd2[0], sd2[1], t2[0], t2[1],
            sd3[0], sd3[1], t3[0], t3[1])

    iu = (colL_src, row_dst)
    ui = (row_src, colL_dst)

    (U1u0a, U1u0b, U1u1a, U1u1b,
     U1u2a, U1u2b, U1ia, U1ib) = seg4(
        iu, (z0a, z0b), iu, (z1a, z1b), iu, (z2a, z2b), ui, (zua, zub))

    pk = lambda a: a.reshape(RP4, 4 * DH)
    (Z1u0a, Z1u0b, Z1u1a, Z1u1b,
     Z1u2a, Z1u2b, Z1ia, Z1ib) = map(lambda a: a.reshape(ROWS_PAD, DH), _mid(
        (pk(U1u0a), pk(U1u0b)), (pk(U1u1a), pk(U1u1b)),
        (pk(U1u2a), pk(U1u2b)), (pk(U1ia), pk(U1ib)),
        su_pk, si_pk))

    (U2ua, U2ub, U2i0a, U2i0b,
     U2i1a, U2i1b, U2i2a, U2i2b) = seg4(
        iu, (Z1ia, Z1ib), ui, (Z1u0a, Z1u0b),
        ui, (Z1u1a, Z1u1b), ui, (Z1u2a, Z1u2b))

    upu, upi = _head_pk(
        (pk(U1u0a), pk(U1u0b), pk(U1u1a), pk(U1u1b),
         pk(U1u2a), pk(U1u2b), pk(U2ua), pk(U2ub)),
        (pk(U1ia), pk(U1ib), pk(U2i0a), pk(U2i0b),
         pk(U2i1a), pk(U2i1b), pk(U2i2a), pk(U2i2b)),
        su_pk, si_pk, _w2(Wu), _w2(Wi))
    user = _head_fin((user_emb, user_emb, user_emb),
                     upu.reshape(ROWS_PAD, D), Wu, bu)
    item = _head_fin((item_emb, v_dense, t_dense),
                     upi.reshape(ROWS_PAD, D), Wi, bi)
    return (user, item)


def kernel(user_emb, item_emb, v_feat, t_feat, Wv, bv, Wt, bt,
           Wu, bu, Wi, bi, edge_index):
    return _pipeline(user_emb, item_emb, v_feat, t_feat, Wv, bv, Wt, bt,
                     Wu, bu, Wi, bi, edge_index)


# speedup vs baseline: 1.2192x; 1.0149x over previous
import functools

import jax
import jax.numpy as jnp
from jax import lax
from jax.experimental import pallas as pl
from jax.experimental.pallas import tpu as pltpu
from jax.experimental.pallas import tpu_sc as plsc

NU = 25000
NI = 25000
NN = NU + NI
D = 64
E = 800000

NC = 2
NS = 16
CH = 128
KS = 8
NSUPER = 50
NJ = NSUPER // 2
NCHUNK = NSUPER * KS
EPT = NCHUNK * CH
EPAD = EPT * NS
NCROW = EPAD // CH
DH = 32
ROWS_PAD = 25088
STRIPE = ROWS_PAD // NS
ABSORB = 25080

EPW = E // (NC * NS)
CNT_WORDS = 51200

_MESH = plsc.VectorSubcoreMesh(
    core_axis_name="c", subcore_axis_name="s", num_cores=NC, num_subcores=NS)


def _wid():
    return lax.axis_index("s") * NC + lax.axis_index("c")


def _deg_body(row_hbm, col_hbm, out_hbm, cnt, idxbuf):
    wid = _wid()
    zeros16 = jnp.zeros((16,), jnp.float32)
    ones16 = jnp.ones((16,), jnp.float32)

    def zero_body(i, c):
        cnt[pl.ds(i * 16, 16)] = zeros16
        return c
    lax.fori_loop(0, CNT_WORDS // 16, zero_body, 0)

    pltpu.sync_copy(row_hbm.at[pl.ds(wid * EPW, EPW)], idxbuf.at[pl.ds(0, EPW)])
    pltpu.sync_copy(col_hbm.at[pl.ds(wid * EPW, EPW)],
                    idxbuf.at[pl.ds(EPW, EPW)])

    def body(i, c):
        iv = idxbuf[pl.ds(i * 16, 16)]
        plsc.addupdate_scatter(cnt, [iv], ones16)
        return c
    lax.fori_loop(0, 2 * EPW // 16, body, 0)

    pltpu.sync_copy(cnt, out_hbm.at[wid])


_deg_kernel = functools.partial(
    pl.kernel,
    out_type=jax.ShapeDtypeStruct((NC * NS, CNT_WORDS), jnp.float32),
    mesh=_MESH,
    compiler_params=pltpu.CompilerParams(needs_layout_passes=False),
    scratch_types=[
        pltpu.VMEM((CNT_WORDS,), jnp.float32),
        pltpu.VMEM((2 * EPW,), jnp.int32),
    ],
)(_deg_body)


def _make_seg_body(offs):
    def _seg_body(zeros_hbm,
                  s0, d0, t0a, t0b, s1, d1, t1a, t1b,
                  s2, d2, t2a, t2b, s3, d3, t3a, t3b,
                  o0a, o0b, o1a, o1b, o2a, o2b, o3a, o3b,
                  acc, sidx, didx, rows, gsem, ssem):
        c = lax.axis_index("c")
        sid = lax.axis_index("s")
        r0 = sid * STRIPE

        def run(src, dst, tab, out, off):
            crow = sid * NCHUNK

            def load_idx(sup, p):
                pltpu.sync_copy(src.at[pl.ds(crow + sup * KS, KS)],
                                sidx.at[p])
                pltpu.sync_copy(dst.at[pl.ds(crow + sup * KS, KS)],
                                didx.at[p])
                if off is not None:
                    for k in range(KS):
                        for i in range(CH // 16):
                            sl = pl.ds(i * 16, 16)
                            sidx[p, k, sl] = sidx[p, k, sl] * 4 + off

            def fire_gathers(p):
                for k in range(KS):
                    pltpu.async_copy(tab.at[sidx.at[p, k]], rows.at[p, k],
                                     gsem)

            def drain_gathers(p):
                for k in range(KS):
                    pltpu.make_async_copy(tab.at[sidx.at[p, k]],
                                          rows.at[p, k], gsem).wait()

            def fire_scatters(p):
                for k in range(KS):
                    pltpu.async_copy(rows.at[p, k], acc.at[didx.at[p, k]],
                                     ssem, add=True)

            def drain_scatters(p):
                for k in range(KS):
                    pltpu.make_async_copy(rows.at[p, k],
                                          acc.at[didx.at[p, k]], ssem).wait()

            load_idx(0, 0)
            fire_gathers(0)
            pltpu.sync_copy(zeros_hbm.at[pl.ds(r0, STRIPE)],
                            acc.at[pl.ds(r0, STRIPE)])
            plsc.subcore_barrier()

            def body(j, carry):
                @pl.when(j > 0)
                def _():
                    drain_scatters(1)
                load_idx(2 * j + 1, 1)
                fire_gathers(1)
                drain_gathers(0)
                fire_scatters(0)
                drain_scatters(0)

                @pl.when(j < NJ - 1)
                def _():
                    load_idx(2 * j + 2, 0)
                    fire_gathers(0)
                drain_gathers(1)
                fire_scatters(1)
                return carry
            lax.fori_loop(0, NJ, body, 0)
            drain_scatters(1)
            plsc.subcore_barrier()
            pltpu.sync_copy(acc.at[pl.ds(r0, STRIPE)],
                            out.at[pl.ds(r0, STRIPE)])

        o = offs if offs is not None else [None] * 8

        @pl.when(c == 0)
        def _():
            run(s0, d0, t0a, o0a, o[0])
            run(s0, d0, t0b, o0b, o[1])
            run(s1, d1, t1a, o1a, o[2])
            run(s1, d1, t1b, o1b, o[3])

        @pl.when(c == 1)
        def _():
            run(s2, d2, t2a, o2a, o[4])
            run(s2, d2, t2b, o2b, o[5])
            run(s3, d3, t3a, o3a, o[6])
            run(s3, d3, t3b, o3b, o[7])

    return _seg_body


_OUT8 = tuple(jax.ShapeDtypeStruct((ROWS_PAD, DH), jnp.float32)
              for _ in range(8))

_seg_partial = functools.partial(
    pl.kernel,
    out_type=_OUT8,
    mesh=_MESH,
    compiler_params=pltpu.CompilerParams(use_tc_tiling_on_sc=False),
    scratch_types=[
        pltpu.MemorySpace.VMEM_SHARED((ROWS_PAD, DH), jnp.float32),
        pltpu.VMEM((2, KS, CH), jnp.int32),
        pltpu.VMEM((2, KS, CH), jnp.int32),
        pltpu.VMEM((2, KS, CH, DH), jnp.float32),
        pltpu.SemaphoreType.DMA,
        pltpu.SemaphoreType.DMA,
    ],
)

_seg_kernel_l1 = _seg_partial(_make_seg_body([2, 3, 0, 1, 2, 3, 0, 1]))
_seg_kernel_l2 = _seg_partial(_make_seg_body(None))


def _deg_finish_body(cnt_ref, dinv_ref):
    c = jnp.sum(cnt_ref[...], axis=0)
    dinv_ref[...] = lax.rsqrt(2.0 * c)


def _deg_finish(cnt):
    return pl.pallas_call(
        _deg_finish_body,
        out_shape=jax.ShapeDtypeStruct((CNT_WORDS // 128, 128), jnp.float32),
    )(cnt)


_RB = 1000


def _proj_body(vf_ref, tf_ref, wv_ref, bv_ref, wt_ref, bt_ref,
               vd_ref, td_ref):
    vf = vf_ref[...]
    tf = tf_ref[...]
    vn = vf * lax.rsqrt(jnp.maximum(jnp.sum(vf * vf, axis=1, keepdims=True),
                                    1e-24))
    tn = tf * lax.rsqrt(jnp.maximum(jnp.sum(tf * tf, axis=1, keepdims=True),
                                    1e-24))
    vd_ref[...] = lax.dot_general(
        vn, wv_ref[...], (((1,), (1,)), ((), ())),
        preferred_element_type=jnp.float32) + bv_ref[...]
    td_ref[...] = lax.dot_general(
        tn, wt_ref[...], (((1,), (1,)), ((), ())),
        preferred_element_type=jnp.float32) + bt_ref[...]


def _proj(v_feat, t_feat, Wv, bv, Wt, bt):
    grid = (NU // _RB,)
    rb = lambda i: (i, 0)
    full = lambda i: (0, 0)
    return pl.pallas_call(
        _proj_body,
        grid=grid,
        in_specs=[
            pl.BlockSpec((_RB, 128), rb), pl.BlockSpec((_RB, 128), rb),
            pl.BlockSpec((D, 128), full), pl.BlockSpec((1, D), full),
            pl.BlockSpec((D, 128), full), pl.BlockSpec((1, D), full),
        ],
        out_specs=(pl.BlockSpec((_RB, D), rb), pl.BlockSpec((_RB, D), rb)),
        out_shape=(jax.ShapeDtypeStruct((NU, D), jnp.float32),
                   jax.ShapeDtypeStruct((NU, D), jnp.float32)),
    )(v_feat, t_feat, Wv, bv, Wt, bt)


def _pre_body(ue_ref, ie_ref, vd_in_ref, td_in_ref, su_ref, si_ref,
              za_ref, zb_ref):
    su = su_ref[...]
    si = si_ref[...]
    za_ref[...] = jnp.concatenate([su * ue_ref[...], si * ie_ref[...]],
                                  axis=1)
    zb_ref[...] = jnp.concatenate([si * vd_in_ref[...], si * td_in_ref[...]],
                                  axis=1)


def _pre(user_emb, item_emb, v_dense, t_dense, s_u, s_i):
    grid = (NU // _RB,)
    rb = lambda i: (i, 0)
    return pl.pallas_call(
        _pre_body,
        grid=grid,
        in_specs=[pl.BlockSpec((_RB, D), rb)] * 4
        + [pl.BlockSpec((_RB, 1), rb)] * 2,
        out_specs=(pl.BlockSpec((_RB, 2 * D), rb),) * 2,
        out_shape=(jax.ShapeDtypeStruct((NU, 2 * D), jnp.float32),) * 2,
    )(user_emb, item_emb, v_dense, t_dense, s_u, s_i)


RP4 = ROWS_PAD // 4
_MB = 1568


def _mid_body(u0a_ref, u0b_ref, u1a_ref, u1b_ref, u2a_ref, u2b_ref,
              uia_ref, uib_ref, su_ref, si_ref,
              z0a_ref, z0b_ref, z1a_ref, z1b_ref, z2a_ref, z2b_ref,
              zia_ref, zib_ref):
    su2 = jnp.square(su_ref[...])
    si2 = jnp.square(si_ref[...])
    z0a_ref[...] = su2 * u0a_ref[...]
    z0b_ref[...] = su2 * u0b_ref[...]
    z1a_ref[...] = su2 * u1a_ref[...]
    z1b_ref[...] = su2 * u1b_ref[...]
    z2a_ref[...] = su2 * u2a_ref[...]
    z2b_ref[...] = su2 * u2b_ref[...]
    zia_ref[...] = si2 * uia_ref[...]
    zib_ref[...] = si2 * uib_ref[...]


def _mid(u0, u1, u2, ui, su_pk, si_pk):
    grid = (RP4 // _MB,)
    rb = lambda i: (i, 0)
    return pl.pallas_call(
        _mid_body,
        grid=grid,
        in_specs=[pl.BlockSpec((_MB, 4 * DH), rb)] * 10,
        out_specs=tuple(pl.BlockSpec((_MB, 4 * DH), rb) for _ in range(8)),
        out_shape=tuple(jax.ShapeDtypeStruct((RP4, 4 * DH), jnp.float32)
                        for _ in range(8)),
    )(*u0, *u1, *u2, *ui, su_pk, si_pk)


_HB = 784


def _head_pk_body(u1u0a, u1u0b, u1u1a, u1u1b, u1u2a, u1u2b, u2ua, u2ub,
                  u1ia, u1ib, u2i0a, u2i0b, u2i1a, u2i1b, u2i2a, u2i2b,
                  su, si, w2u, w2i, outu, outi):
    third = 1.0 / 3.0
    s = su[...] * third
    mu = jnp.concatenate([
        s * (u1u0a[...] + u2ua[...]), s * (u1u0b[...] + u2ub[...]),
        s * (u1u1a[...] + u2ua[...]), s * (u1u1b[...] + u2ub[...]),
        s * (u1u2a[...] + u2ua[...]), s * (u1u2b[...] + u2ub[...]),
    ], axis=1)
    outu[...] = lax.dot_general(mu, w2u[...], (((1,), (0,)), ((), ())),
                                preferred_element_type=jnp.float32)
    t = si[...] * third
    mi = jnp.concatenate([
        t * (u1ia[...] + u2i0a[...]), t * (u1ib[...] + u2i0b[...]),
        t * (u1ia[...] + u2i1a[...]), t * (u1ib[...] + u2i1b[...]),
        t * (u1ia[...] + u2i2a[...]), t * (u1ib[...] + u2i2b[...]),
    ], axis=1)
    outi[...] = lax.dot_general(mi, w2i[...], (((1,), (0,)), ((), ())),
                                preferred_element_type=jnp.float32)


def _head_pk(uhalves, ihalves, su_pk, si_pk, W2u, W2i):
    grid = (RP4 // _HB,)
    rb = lambda i: (i, 0)
    full = lambda i: (0, 0)
    return pl.pallas_call(
        _head_pk_body,
        grid=grid,
        in_specs=[pl.BlockSpec((_HB, 4 * DH), rb)] * 18
        + [pl.BlockSpec((6 * 4 * DH, 4 * D), full)] * 2,
        out_specs=(pl.BlockSpec((_HB, 4 * D), rb),) * 2,
        out_shape=(jax.ShapeDtypeStruct((RP4, 4 * D), jnp.float32),) * 2,
    )(*uhalves, *ihalves, su_pk, si_pk, W2u, W2i)


def _head_fin_body(x0, x1, x2, up, w, b, out):
    m = jnp.concatenate([x0[...], x1[...], x2[...]], axis=1)
    out[...] = lax.dot_general(
        m, w[...], (((1,), (1,)), ((), ())),
        preferred_element_type=jnp.float32) * (1.0 / 3.0) + up[...] + b[...]


def _head_fin(xs, up, W, b):
    grid = (NU // _RB,)
    rb = lambda i: (i, 0)
    full = lambda i: (0, 0)
    return pl.pallas_call(
        _head_fin_body,
        grid=grid,
        in_specs=[pl.BlockSpec((_RB, D), rb)] * 4
        + [pl.BlockSpec((D, 3 * D), full), pl.BlockSpec((1, D), full)],
        out_specs=pl.BlockSpec((_RB, D), rb),
        out_shape=jax.ShapeDtypeStruct((NU, D), jnp.float32),
    )(*xs, up, W, b)


def _w2(W):
    WT6 = W.T.reshape(6, DH, D)
    eye4 = jnp.eye(4, dtype=W.dtype)
    A = WT6[:, None, :, None, :] * eye4[None, :, None, :, None]
    return A.reshape(6 * 4 * DH, 4 * D)


def _pipeline(user_emb, item_emb, v_feat, t_feat, Wv, bv, Wt, bt,
              Wu, bu, Wi, bi, edge_index):
    row = edge_index[0]
    col = edge_index[1]
    colL = col - NU
    bv = bv.reshape(1, D)
    bt = bt.reshape(1, D)
    bu = bu.reshape(1, D)
    bi = bi.reshape(1, D)

    pad_src = jnp.zeros((EPAD - E,), jnp.int32)
    pad_dst = jnp.full((EPAD - E,), ABSORB, jnp.int32)
    row_src = jnp.concatenate([row, pad_src]).reshape(NCROW, CH)
    row_dst = jnp.concatenate([row, pad_dst]).reshape(NCROW, CH)
    colL_src = jnp.concatenate([colL, pad_src]).reshape(NCROW, CH)
    colL_dst = jnp.concatenate([colL, pad_dst]).reshape(NCROW, CH)

    v_dense, t_dense = _proj(v_feat, t_feat, Wv, bv, Wt, bt)

    cnt = _deg_kernel(row, col)
    dinv = _deg_finish(cnt.reshape(NC * NS, CNT_WORDS // 128, 128))
    dflat = dinv.reshape(-1)
    s_u = dflat[:NU].reshape(NU, 1)
    s_i = dflat[NU:NN].reshape(NI, 1)
    su_pk = jnp.broadcast_to(dflat[:ROWS_PAD, None],
                             (ROWS_PAD, DH)).reshape(RP4, 4 * DH)
    si_pk = jnp.broadcast_to(
        lax.dynamic_slice(dflat, (NU,), (ROWS_PAD,))[:, None],
        (ROWS_PAD, DH)).reshape(RP4, 4 * DH)

    ZA, ZB = _pre(user_emb, item_emb, v_dense, t_dense, s_u, s_i)
    ZAv = ZA.reshape(4 * NU, DH)
    ZBv = ZB.reshape(4 * NU, DH)

    zeros = jnp.zeros((ROWS_PAD, DH), jnp.float32)

    iu = (colL_src, row_dst)
    ui = (row_src, colL_dst)

    (U1u0a, U1u0b, U1u1a, U1u1b,
     U1u2a, U1u2b, U1ia, U1ib) = _seg_kernel_l1(
        zeros,
        iu[0], iu[1], ZAv, ZAv,
        iu[0], iu[1], ZBv, ZBv,
        iu[0], iu[1], ZBv, ZBv,
        ui[0], ui[1], ZAv, ZAv)

    pk = lambda a: a.reshape(RP4, 4 * DH)
    (Z1u0a, Z1u0b, Z1u1a, Z1u1b,
     Z1u2a, Z1u2b, Z1ia, Z1ib) = map(lambda a: a.reshape(ROWS_PAD, DH), _mid(
        (pk(U1u0a), pk(U1u0b)), (pk(U1u1a), pk(U1u1b)),
        (pk(U1u2a), pk(U1u2b)), (pk(U1ia), pk(U1ib)),
        su_pk, si_pk))

    (U2ua, U2ub, U2i0a, U2i0b,
     U2i1a, U2i1b, U2i2a, U2i2b) = _seg_kernel_l2(
        zeros,
        iu[0], iu[1], Z1ia, Z1ib,
        ui[0], ui[1], Z1u0a, Z1u0b,
        ui[0], ui[1], Z1u1a, Z1u1b,
        ui[0], ui[1], Z1u2a, Z1u2b)

    upu, upi = _head_pk(
        (pk(U1u0a), pk(U1u0b), pk(U1u1a), pk(U1u1b),
         pk(U1u2a), pk(U1u2b), pk(U2ua), pk(U2ub)),
        (pk(U1ia), pk(U1ib), pk(U2i0a), pk(U2i0b),
         pk(U2i1a), pk(U2i1b), pk(U2i2a), pk(U2i2b)),
        su_pk, si_pk, _w2(Wu), _w2(Wi))
    user = _head_fin((user_emb, user_emb, user_emb),
                     upu.reshape(ROWS_PAD, D), Wu, bu)
    item = _head_fin((item_emb, v_dense, t_dense),
                     upi.reshape(ROWS_PAD, D), Wi, bi)
    return (user, item)


def kernel(user_emb, item_emb, v_feat, t_feat, Wv, bv, Wt, bt,
           Wu, bu, Wi, bi, edge_index):
    return _pipeline(user_emb, item_emb, v_feat, t_feat, Wv, bv, Wt, bt,
                     Wu, bu, Wi, bi, edge_index)
